# Initial kernel scaffold; baseline (speedup 1.0000x reference)
import jax
import jax.numpy as jnp
from jax.experimental import pallas as pl


def kernel(wid, edge_index, tree_id, tree_vec, m0, rm0, p_targets, emb, Wz, bz, Wh, bh, Wr, Ur, bur, Wl, bl, Wo, bo, Uu, bu, Us, bs):
    raise NotImplementedError("write your pallas kernel here")


# speedup vs baseline: 1.7185x; 1.7185x over previous
import functools

import jax
import jax.numpy as jnp
from jax import lax
from jax.experimental import pallas as pl
from jax.experimental.pallas import tpu as pltpu
from jax.experimental.pallas import tpu_sc as plsc

N = 10000
E = 320000
H = 128
L = 128
V = 1000
T = 256

NC = 2
NS = 16
NW = NC * NS
CH = 80

N_PAD = 10240
NA = 10240

_mesh = lambda: plsc.VectorSubcoreMesh(
    core_axis_name="c", subcore_axis_name="s", num_cores=NC, num_subcores=NS)


def _worker_id():
  return lax.axis_index("s") * NC + lax.axis_index("c")


def _sc_gather_nodes_body(emb, embA, embB, embC, tvec, wid, tid,
                          x, a, b, c, tv, idx_w, idx_t, buf, sem):
  w = _worker_id()
  per_w = N_PAD // NW
  n_ch = per_w // CH

  def chunk(i, _):
    base = w * per_w + i * CH
    pltpu.sync_copy(wid.at[pl.ds(base, CH)], idx_w)
    pltpu.sync_copy(tid.at[pl.ds(base, CH)], idx_t)
    for tab, out, idx in ((emb, x, idx_w), (embA, a, idx_w),
                          (embB, b, idx_w), (embC, c, idx_w),
                          (tvec, tv, idx_t)):
      pltpu.async_copy(tab.at[idx], buf, sem).wait()
      pltpu.sync_copy(buf, out.at[pl.ds(base, CH)])
    return ()

  lax.fori_loop(0, n_ch, chunk, (), unroll=False)


def _sc_gather_nodes(emb, embA, embB, embC, tvec, wid, tid):
  f = pl.kernel(
      _sc_gather_nodes_body,
      out_type=[jax.ShapeDtypeStruct((N_PAD, H), jnp.float32)] * 5,
      mesh=_mesh(),
      scratch_types=[
          pltpu.VMEM((CH,), jnp.int32),
          pltpu.VMEM((CH,), jnp.int32),
          pltpu.VMEM((CH, H), jnp.float32),
          pltpu.SemaphoreType.DMA,
      ],
  )
  return f(emb, embA, embB, embC, tvec, wid, tid)


def _sc_scatter2_body(m, rm, dst, zeros, nm, nrm, idx_v, rows, acc):
  c = lax.axis_index("c")
  s = lax.axis_index("s")
  rows_per_tile = NA // NS
  r0 = s * rows_per_tile
  pltpu.sync_copy(zeros.at[pl.ds(r0, rows_per_tile)],
                  acc.at[pl.ds(r0, rows_per_tile)])
  plsc.subcore_barrier()

  per_tile = E // NS
  n_ch = per_tile // CH

  def run(tab):
    def chunk(i, _):
      base = s * per_tile + i * CH
      pltpu.sync_copy(dst.at[pl.ds(base, CH)], idx_v)
      pltpu.sync_copy(tab.at[pl.ds(base, CH)], rows)
      pltpu.sync_copy(rows, acc.at[idx_v], add=True)
      return ()
    lax.fori_loop(0, n_ch, chunk, (), unroll=False)

  @pl.when(c == 0)
  def _():
    run(m)

  @pl.when(c == 1)
  def _():
    run(rm)

  plsc.subcore_barrier()

  @pl.when(c == 0)
  def _():
    pltpu.sync_copy(acc.at[pl.ds(r0, rows_per_tile)],
                    nm.at[pl.ds(r0, rows_per_tile)])

  @pl.when(c == 1)
  def _():
    pltpu.sync_copy(acc.at[pl.ds(r0, rows_per_tile)],
                    nrm.at[pl.ds(r0, rows_per_tile)])


def _sc_scatter2(m, rm, dst, zeros):
  f = pl.kernel(
      _sc_scatter2_body,
      out_type=[jax.ShapeDtypeStruct((NA, H), jnp.float32)] * 2,
      mesh=_mesh(),
      scratch_types=[
          pltpu.VMEM((CH,), jnp.int32),
          pltpu.VMEM((CH, H), jnp.float32),
          pltpu.VMEM_SHARED((NA, H), jnp.float32),
      ],
  )
  return f(m, rm, dst, zeros)


def _sc_scatter_half_body(m, dst, zeros, hp, idx_v, rows, acc):
  c = lax.axis_index("c")
  s = lax.axis_index("s")
  rows_per_tile = NA // NS
  r0 = s * rows_per_tile
  pltpu.sync_copy(zeros.at[pl.ds(r0, rows_per_tile)],
                  acc.at[pl.ds(r0, rows_per_tile)])
  plsc.subcore_barrier()

  per_tile = (E // NC) // NS
  n_ch = per_tile // CH

  def chunk(i, _):
    base = c * (E // NC) + s * per_tile + i * CH
    pltpu.sync_copy(dst.at[pl.ds(base, CH)], idx_v)
    pltpu.sync_copy(m.at[pl.ds(base, CH)], rows)
    pltpu.sync_copy(rows, acc.at[idx_v], add=True)
    return ()

  lax.fori_loop(0, n_ch, chunk, (), unroll=False)
  plsc.subcore_barrier()
  pltpu.sync_copy(acc.at[pl.ds(r0, rows_per_tile)],
                  hp.at[c, pl.ds(r0, rows_per_tile)])


def _sc_scatter_half(m, dst, zeros):
  f = pl.kernel(
      _sc_scatter_half_body,
      out_type=jax.ShapeDtypeStruct((NC, NA, H), jnp.float32),
      mesh=_mesh(),
      scratch_types=[
          pltpu.VMEM((CH,), jnp.int32),
          pltpu.VMEM((CH, H), jnp.float32),
          pltpu.VMEM_SHARED((NA, H), jnp.float32),
      ],
  )
  return f(m, dst, zeros)


def _sc_gather_edges_body(tabs_src, tab_dst, src, dst, outs_src, out_dst,
                          idx_s, idx_d, buf, sem):
  w = _worker_id()
  per_w = E // NW
  n_ch = per_w // CH

  def chunk(i, _):
    base = w * per_w + i * CH
    pltpu.sync_copy(src.at[pl.ds(base, CH)], idx_s)
    for tab, out in zip(tabs_src, outs_src):
      pltpu.async_copy(tab.at[idx_s], buf, sem).wait()
      pltpu.sync_copy(buf, out.at[pl.ds(base, CH)])
    if tab_dst is not None:
      pltpu.sync_copy(dst.at[pl.ds(base, CH)], idx_d)
      pltpu.async_copy(tab_dst.at[idx_d], buf, sem).wait()
      pltpu.sync_copy(buf, out_dst.at[pl.ds(base, CH)])
    return ()

  lax.fori_loop(0, n_ch, chunk, (), unroll=False)


def _sc_gather_edges5(nm, nrm, a, b, c, src, dst):
  def body(nm, nrm, a, b, c, src, dst, g0, g1, g2, g3, g4,
           idx_s, idx_d, buf, sem):
    _sc_gather_edges_body((nm, nrm, a, b), c, src, dst,
                          (g0, g1, g2, g3), g4, idx_s, idx_d, buf, sem)

  f = pl.kernel(
      body,
      out_type=[jax.ShapeDtypeStruct((E, H), jnp.float32)] * 5,
      mesh=_mesh(),
      scratch_types=[
          pltpu.VMEM((CH,), jnp.int32),
          pltpu.VMEM((CH,), jnp.int32),
          pltpu.VMEM((CH, H), jnp.float32),
          pltpu.SemaphoreType.DMA,
      ],
  )
  return f(nm, nrm, a, b, c, src, dst)


def _sc_gather_edges2(nm, nrm, src):
  def body(nm, nrm, src, g0, g1, idx_s, idx_d, buf, sem):
    _sc_gather_edges_body((nm, nrm), None, src, None,
                          (g0, g1), None, idx_s, idx_d, buf, sem)

  f = pl.kernel(
      body,
      out_type=[jax.ShapeDtypeStruct((E, H), jnp.float32)] * 2,
      mesh=_mesh(),
      scratch_types=[
          pltpu.VMEM((CH,), jnp.int32),
          pltpu.VMEM((CH,), jnp.int32),
          pltpu.VMEM((CH, H), jnp.float32),
          pltpu.SemaphoreType.DMA,
      ],
  )
  return f(nm, nrm, src)


def _tc_prep_body(emb_ref, wz1_ref, wh1_ref, wr_ref, bz_ref, bh_ref, bur_ref,
                  ea_ref, eb_ref, ec_ref):
  emb = emb_ref[...]
  ea_ref[...] = jnp.dot(emb, wz1_ref[...],
                        preferred_element_type=jnp.float32) + bz_ref[...]
  eb_ref[...] = jnp.dot(emb, wh1_ref[...],
                        preferred_element_type=jnp.float32) + bh_ref[...]
  ec_ref[...] = jnp.dot(emb, wr_ref[...],
                        preferred_element_type=jnp.float32) + bur_ref[...]


def _tc_prep(emb, Wz1, Wh1, Wr, bz, bh, bur):
  return pl.pallas_call(
      _tc_prep_body,
      out_shape=[jax.ShapeDtypeStruct((V, H), jnp.float32)] * 3,
  )(emb, Wz1, Wh1, Wr, bz, bh, bur)


EB = 512


def _pair_swap(x):
  nxt = pltpu.roll(x, x.shape[0] - 1, 0)
  prv = pltpu.roll(x, 1, 0)
  row = lax.broadcasted_iota(jnp.int32, x.shape, 0)
  return jnp.where((row & 1) == 0, nxt, prv)


def _tc_gru_body(m_ref, rm_ref, gnm_ref, gnrm_ref, pz_ref, ph_ref, pr_ref,
                 wz2_ref, wh2_ref, ur_ref, mo_ref, rmo_ref):
  s = gnm_ref[...] - _pair_swap(m_ref[...])
  arm = gnrm_ref[...] - _pair_swap(rm_ref[...])
  z = jax.nn.sigmoid(
      pz_ref[...] + jnp.dot(s, wz2_ref[...],
                            preferred_element_type=jnp.float32))
  mt = jnp.tanh(
      ph_ref[...] + jnp.dot(arm, wh2_ref[...],
                            preferred_element_type=jnp.float32))
  mo = (1.0 - z) * s + z * mt
  r = jax.nn.sigmoid(
      pr_ref[...] + jnp.dot(mo, ur_ref[...],
                            preferred_element_type=jnp.float32))
  mo_ref[...] = mo
  rmo_ref[...] = r * mo


def _tc_gru(m, rm, gnm, gnrm, pz, ph, pr, Wz2, Wh2, Ur):
  eb_spec = pl.BlockSpec((EB, H), lambda i: (i, 0))
  w_spec = pl.BlockSpec((H, H), lambda i: (0, 0))
  return pl.pallas_call(
      _tc_gru_body,
      grid=(E // EB,),
      in_specs=[eb_spec] * 7 + [w_spec] * 3,
      out_specs=[eb_spec] * 2,
      out_shape=[jax.ShapeDtypeStruct((E, H), jnp.float32)] * 2,
  )(m, rm, gnm, gnrm, pz, ph, pr, Wz2, Wh2, Ur)


NB = 400
VP = 1024


def _tc_final_body(x_ref, h0_ref, h1_ref, tv_ref, widf_ref, pt_ref,
                   wl1_ref, wl2_ref, blr, wo_ref, bo_ref,
                   uu1_ref, uu2_ref, uu3_ref, bur2, us_ref, bsr,
                   acc_ref):
  pid = pl.program_id(0)

  @pl.when(pid == 0)
  def _():
    for i in range(4):
      acc_ref[i] = 0.0

  h = h0_ref[...] + h1_ref[...]
  x = x_ref[...]
  tv = tv_ref[...]

  qp = jax.nn.relu(
      jnp.dot(h, wl1_ref[...], preferred_element_type=jnp.float32)
      + jnp.dot(tv, wl2_ref[...], preferred_element_type=jnp.float32)
      + blr[...])
  q = jnp.dot(qp, wo_ref[...], preferred_element_type=jnp.float32) + bo_ref[...]

  pp = jax.nn.relu(
      jnp.dot(x, uu1_ref[...], preferred_element_type=jnp.float32)
      + jnp.dot(h, uu2_ref[...], preferred_element_type=jnp.float32)
      + jnp.dot(tv, uu3_ref[...], preferred_element_type=jnp.float32)
      + bur2[...])
  p = jnp.dot(pp, us_ref[...], preferred_element_type=jnp.float32) + bsr[...]

  pt = pt_ref[...]
  p_loss = jnp.sum(jnp.maximum(p, 0.0) - p * pt
                   + jnp.log(1.0 + jnp.exp(-jnp.abs(p))))
  p_hit = jnp.sum(jnp.where((p > 0.0) == (pt > 0.5), 1.0, 0.0))

  widf = widf_ref[...]
  coli = lax.broadcasted_iota(jnp.int32, q.shape, 1)
  col = coli.astype(jnp.float32)
  onehot = jnp.where(col == widf, 1.0, 0.0)
  q_sel = jnp.sum(q * onehot, axis=1, keepdims=True)
  q_max = jnp.max(q, axis=1, keepdims=True)
  lse = jnp.log(jnp.sum(jnp.exp(q - q_max), axis=1, keepdims=True)) + q_max
  q_loss = jnp.sum(lse - q_sel)

  am = jnp.min(jnp.where(q == q_max, coli, VP), axis=1, keepdims=True)
  q_hit = jnp.sum(jnp.where(am.astype(jnp.float32) == widf, 1.0, 0.0))

  acc_ref[0] += q_loss
  acc_ref[1] += p_loss
  acc_ref[2] += q_hit
  acc_ref[3] += p_hit


def _tc_final(x, h2, tv, widf, pt, Wl, bl, Wo_p, bo_p, Uu, bu, Us, bs):
  nb_spec = pl.BlockSpec((NB, H), lambda i: (i, 0))
  n1_spec = pl.BlockSpec((NB, 1), lambda i: (i, 0))
  full = lambda shape: pl.BlockSpec(shape, lambda i: tuple(0 for _ in shape))
  return pl.pallas_call(
      _tc_final_body,
      grid=(N // NB,),
      in_specs=[nb_spec, nb_spec, nb_spec, nb_spec, n1_spec, n1_spec,
                full((H, H)), full((H, H)), full((1, H)),
                full((H, VP)), full((1, VP)),
                full((H, H)), full((H, H)), full((H, H)), full((1, H)),
                full((H, 1)), full((1, 1))],
      out_specs=pl.BlockSpec(memory_space=pltpu.MemorySpace.SMEM),
      out_shape=jax.ShapeDtypeStruct((4,), jnp.float32),
  )(x, h2[0], h2[1], tv, widf, pt,
    Wl[:H], Wl[H:], bl.reshape(1, H), Wo_p, bo_p,
    Uu[:H], Uu[H:2 * H], Uu[2 * H:], bu.reshape(1, H), Us, bs.reshape(1, 1))


def kernel(wid, edge_index, tree_id, tree_vec, m0, rm0, p_targets, emb,
           Wz, bz, Wh, bh, Wr, Ur, bur, Wl, bl, Wo, bo, Uu, bu, Us, bs):
  src = edge_index[0].astype(jnp.int32)
  dst = edge_index[1].astype(jnp.int32)
  wid_p = jnp.concatenate(
      [wid.astype(jnp.int32), jnp.zeros((N_PAD - N,), jnp.int32)])
  tid_p = jnp.concatenate(
      [tree_id.astype(jnp.int32), jnp.zeros((N_PAD - N,), jnp.int32)])
  zeros_n = jnp.zeros((NA, H), jnp.float32)

  embA, embB, embC = _tc_prep(emb, Wz[:H], Wh[:H], Wr,
                              bz.reshape(1, H), bh.reshape(1, H),
                              bur.reshape(1, H))

  x_p, a_p, b_p, c_p, tv_p = _sc_gather_nodes(
      emb, embA, embB, embC, tree_vec, wid_p, tid_p)

  nm, nrm = _sc_scatter2(m0, rm0, dst, zeros_n)
  gnm, gnrm, pz, ph, pr = _sc_gather_edges5(nm, nrm, a_p, b_p, c_p, src, dst)
  m1, rm1 = _tc_gru(m0, rm0, gnm, gnrm, pz, ph, pr, Wz[H:], Wh[H:], Ur)

  nm2, nrm2 = _sc_scatter2(m1, rm1, dst, zeros_n)
  gnm2, gnrm2 = _sc_gather_edges2(nm2, nrm2, src)
  m2, _ = _tc_gru(m1, rm1, gnm2, gnrm2, pz, ph, pr, Wz[H:], Wh[H:], Ur)

  h2 = _sc_scatter_half(m2, dst, zeros_n)

  Wo_p = jnp.concatenate([Wo, jnp.zeros((H, VP - V), jnp.float32)], axis=1)
  bo_p = jnp.concatenate([bo, jnp.full((VP - V,), -1e9, jnp.float32)])
  widf = wid.astype(jnp.float32).reshape(N, 1)
  ptf = p_targets.astype(jnp.float32).reshape(N, 1)

  acc = _tc_final(x_p[:N], h2[:, :N], tv_p[:N], widf, ptf,
                  Wl, bl, Wo_p, bo_p.reshape(1, VP), Uu, bu, Us, bs)

  n_trees = float(T)
  q_loss = acc[0] / n_trees
  p_loss = acc[1] / n_trees
  q_acc = acc[2] / float(N)
  p_acc = acc[3] / float(N)
  return q_loss, p_loss, q_acc, p_acc


# speedup vs baseline: 2.5491x; 1.4833x over previous
import jax
import jax.numpy as jnp
from jax import lax
from jax.experimental import pallas as pl
from jax.experimental.pallas import tpu as pltpu
from jax.experimental.pallas import tpu_sc as plsc

N = 10000
E = 320000
H = 128
H2 = 2 * H
L = 128
V = 1000
T = 256

NC = 2
NS = 16
NW = NC * NS
CH = 80
MEG = 2

N_PAD = 10240
NA = 10112

_mesh = lambda: plsc.VectorSubcoreMesh(
    core_axis_name="c", subcore_axis_name="s", num_cores=NC, num_subcores=NS)


def _worker_id():
  return lax.axis_index("s") * NC + lax.axis_index("c")


def _drain(dummy_src, dst, sem):
  pltpu.make_async_copy(dummy_src, dst, sem).wait()


def _sc_gather_nodes_body(emb, ab2, c2, tvec, wid, tid,
                          x, ab, c, tv, idx_w, idx_t, bx, bab, bc, btv, sem):
  w = _worker_id()
  per_w = N_PAD // NW
  n_ch = per_w // CH

  def chunk(i, _):
    base = w * per_w + i * CH
    pltpu.sync_copy(wid.at[pl.ds(base, CH)], idx_w)
    pltpu.sync_copy(tid.at[pl.ds(base, CH)], idx_t)
    pltpu.async_copy(emb.at[idx_w], bx, sem)
    pltpu.async_copy(ab2.at[idx_w], bab, sem)
    pltpu.async_copy(c2.at[idx_w], bc, sem)
    pltpu.async_copy(tvec.at[idx_t], btv, sem)
    _drain(emb.at[pl.ds(0, CH)], bx, sem)
    _drain(ab2.at[pl.ds(0, CH)], bab, sem)
    _drain(c2.at[pl.ds(0, CH)], bc, sem)
    _drain(tvec.at[pl.ds(0, CH)], btv, sem)
    pltpu.sync_copy(bx, x.at[pl.ds(base, CH)])
    pltpu.sync_copy(bab, ab.at[pl.ds(base, CH)])
    pltpu.sync_copy(bc, c.at[pl.ds(base, CH)])
    pltpu.sync_copy(btv, tv.at[pl.ds(base, CH)])
    return ()

  lax.fori_loop(0, n_ch, chunk, (), unroll=False)


def _sc_gather_nodes(emb, ab2, c2, tvec, wid, tid):
  f = pl.kernel(
      _sc_gather_nodes_body,
      out_type=[jax.ShapeDtypeStruct((N_PAD, H), jnp.float32),
                jax.ShapeDtypeStruct((N_PAD, H2), jnp.float32),
                jax.ShapeDtypeStruct((N_PAD, H), jnp.float32),
                jax.ShapeDtypeStruct((N_PAD, H), jnp.float32)],
      mesh=_mesh(),
      scratch_types=[
          pltpu.VMEM((CH,), jnp.int32),
          pltpu.VMEM((CH,), jnp.int32),
          pltpu.VMEM((CH, H), jnp.float32),
          pltpu.VMEM((CH, H2), jnp.float32),
          pltpu.VMEM((CH, H), jnp.float32),
          pltpu.VMEM((CH, H), jnp.float32),
          pltpu.SemaphoreType.DMA,
      ],
  )
  return f(emb, ab2, c2, tvec, wid, tid)


def _scatter_accumulate(tab, dst3, acc, tile_mg0, n_meg, idx2, rows, sems):
  sem_l, sem_a = sems

  def load(mi, b):
    mg = tile_mg0 + mi
    pltpu.async_copy(dst3.at[mg], idx2[b], sem_l[b])
    pltpu.async_copy(tab.at[pl.ds(mg * MEG * CH, MEG * CH)], rows[b], sem_l[b])

  def wait_load(b):
    _drain(dst3.at[0], idx2[b], sem_l[b])
    _drain(tab.at[pl.ds(0, MEG * CH)], rows[b], sem_l[b])

  def fire_adds(b):
    for j in range(MEG):
      pltpu.async_copy(rows[b].at[pl.ds(j * CH, CH)], acc.at[idx2[b].at[j]],
                       sem_a[b], add=True)

  def wait_adds(b):
    _drain(tab.at[pl.ds(0, MEG * CH)], rows[b], sem_a[b])

  load(0, 0)

  def step(mi, _):
    for b in (0, 1):
      m = 2 * mi + b

      @pl.when(m < n_meg)
      def _():
        wait_load(b)
        fire_adds(b)

        @pl.when(m + 1 < n_meg)
        def _():
          load(m + 1, 1 - b)

        wait_adds(b)
    return ()

  lax.fori_loop(0, (n_meg + 1) // 2, step, (), unroll=False)


def _sc_scatter2_body(m, rm, dst3, zeros, g, idx2a, idx2b, rowsa, rowsb, acc,
                      sl0, sl1, sa0, sa1):
  c = lax.axis_index("c")
  s = lax.axis_index("s")
  rows_per_tile = NA // NS
  r0 = s * rows_per_tile
  pltpu.sync_copy(zeros.at[pl.ds(r0, rows_per_tile)],
                  acc.at[pl.ds(r0, rows_per_tile)])
  plsc.subcore_barrier()

  per_tile_meg = (E // NS) // (MEG * CH)
  sems = ((sl0, sl1), (sa0, sa1))

  @pl.when(c == 0)
  def _():
    _scatter_accumulate(m, dst3, acc, s * per_tile_meg, per_tile_meg,
                        (idx2a, idx2b), (rowsa, rowsb), sems)

  @pl.when(c == 1)
  def _():
    _scatter_accumulate(rm, dst3, acc, s * per_tile_meg, per_tile_meg,
                        (idx2a, idx2b), (rowsa, rowsb), sems)

  plsc.subcore_barrier()

  @pl.when(c == 0)
  def _():
    pltpu.sync_copy(acc.at[pl.ds(r0, rows_per_tile)],
                    g.at[pl.ds(r0, rows_per_tile), pl.ds(0, H)])

  @pl.when(c == 1)
  def _():
    pltpu.sync_copy(acc.at[pl.ds(r0, rows_per_tile)],
                    g.at[pl.ds(r0, rows_per_tile), pl.ds(H, H)])


def _sc_scatter2(m, rm, dst3, zeros):
  f = pl.kernel(
      _sc_scatter2_body,
      out_type=jax.ShapeDtypeStruct((NA, H2), jnp.float32),
      mesh=_mesh(),
      scratch_types=[
          pltpu.VMEM((MEG, CH), jnp.int32),
          pltpu.VMEM((MEG, CH), jnp.int32),
          pltpu.VMEM((MEG * CH, H), jnp.float32),
          pltpu.VMEM((MEG * CH, H), jnp.float32),
          pltpu.VMEM_SHARED((NA, H), jnp.float32),
          pltpu.SemaphoreType.DMA,
          pltpu.SemaphoreType.DMA,
          pltpu.SemaphoreType.DMA,
          pltpu.SemaphoreType.DMA,
      ],
  )
  return f(m, rm, dst3, zeros)


def _sc_scatter_half_body(m, dst3, zeros, hp, idx2a, idx2b, rowsa, rowsb, acc,
                          sl0, sl1, sa0, sa1):
  c = lax.axis_index("c")
  s = lax.axis_index("s")
  rows_per_tile = NA // NS
  r0 = s * rows_per_tile
  pltpu.sync_copy(zeros.at[pl.ds(r0, rows_per_tile)],
                  acc.at[pl.ds(r0, rows_per_tile)])
  plsc.subcore_barrier()

  sems = ((sl0, sl1), (sa0, sa1))

  @pl.when(c == 0)
  def _():
    _scatter_accumulate(m, dst3, acc, s * 64, 64,
                        (idx2a, idx2b), (rowsa, rowsb), sems)

  @pl.when(c == 1)
  def _():
    _scatter_accumulate(m, dst3, acc, 1024 + s * 61, 61,
                        (idx2a, idx2b), (rowsa, rowsb), sems)

  plsc.subcore_barrier()
  pltpu.sync_copy(acc.at[pl.ds(r0, rows_per_tile)],
                  hp.at[c, pl.ds(r0, rows_per_tile)])


def _sc_scatter_half(m, dst3, zeros):
  f = pl.kernel(
      _sc_scatter_half_body,
      out_type=jax.ShapeDtypeStruct((NC, NA, H), jnp.float32),
      mesh=_mesh(),
      scratch_types=[
          pltpu.VMEM((MEG, CH), jnp.int32),
          pltpu.VMEM((MEG, CH), jnp.int32),
          pltpu.VMEM((MEG * CH, H), jnp.float32),
          pltpu.VMEM((MEG * CH, H), jnp.float32),
          pltpu.VMEM_SHARED((NA, H), jnp.float32),
          pltpu.SemaphoreType.DMA,
          pltpu.SemaphoreType.DMA,
          pltpu.SemaphoreType.DMA,
          pltpu.SemaphoreType.DMA,
      ],
  )
  return f(m, dst3, zeros)


def _gather_pipeline(streams, src1, dst1, outs, idx_s, idx_d, bufs, sem_g,
                     sem_w):
  w = _worker_id()
  per_w = E // NW
  n_ch = per_w // CH
  any_dst = any(use_d for _, _, use_d in streams)

  def fire_gathers(ch, b):
    base = w * per_w + ch * CH
    pltpu.sync_copy(src1.at[pl.ds(base, CH)], idx_s[b])
    if any_dst:
      pltpu.sync_copy(dst1.at[pl.ds(base, CH)], idx_d[b])
    for k, (tab, _, use_d) in enumerate(streams):
      idx = idx_d[b] if use_d else idx_s[b]
      pltpu.async_copy(tab.at[idx], bufs[k][b], sem_g[b])

  def wait_gathers(b):
    for k, (tab, _, _) in enumerate(streams):
      _drain(tab.at[pl.ds(0, CH)], bufs[k][b], sem_g[b])

  def fire_wb(ch, b):
    base = w * per_w + ch * CH
    for k in range(len(streams)):
      pltpu.async_copy(bufs[k][b], outs[k].at[pl.ds(base, CH)], sem_w[b])

  def wait_wb(b):
    for k, (tab, _, _) in enumerate(streams):
      _drain(tab.at[pl.ds(0, CH)], bufs[k][b], sem_w[b])

  fire_gathers(0, 0)

  def step(i, _):
    for b in (0, 1):
      ch = 2 * i + b
      wait_gathers(b)
      fire_wb(ch, b)

      @pl.when(ch >= 1)
      def _():
        wait_wb(1 - b)

      @pl.when(ch + 1 < n_ch)
      def _():
        fire_gathers(ch + 1, 1 - b)
    return ()

  lax.fori_loop(0, n_ch // 2, step, (), unroll=False)
  b_last = (n_ch - 1) % 2
  wait_gathers(b_last)
  fire_wb(n_ch - 1, b_last)
  wait_wb(1 - b_last)
  wait_wb(b_last)


def _sc_gather_edges3(g_tab, ab_tab, c_tab, src1, dst1):
  def body(g_tab, ab_tab, c_tab, src1, dst1, og, oab, oc,
           i_s0, i_s1, i_d0, i_d1, bg0, bg1, bab0, bab1, bc0, bc1,
           sg0, sg1, sw0, sw1):
    _gather_pipeline(
        [(g_tab, H2, False), (ab_tab, H2, False), (c_tab, H, True)],
        src1, dst1, (og, oab, oc),
        (i_s0, i_s1), (i_d0, i_d1),
        ((bg0, bg1), (bab0, bab1), (bc0, bc1)),
        (sg0, sg1), (sw0, sw1))

  f = pl.kernel(
      body,
      out_type=[jax.ShapeDtypeStruct((E, H2), jnp.float32),
                jax.ShapeDtypeStruct((E, H2), jnp.float32),
                jax.ShapeDtypeStruct((E, H), jnp.float32)],
      mesh=_mesh(),
      scratch_types=[
          pltpu.VMEM((CH,), jnp.int32), pltpu.VMEM((CH,), jnp.int32),
          pltpu.VMEM((CH,), jnp.int32), pltpu.VMEM((CH,), jnp.int32),
          pltpu.VMEM((CH, H2), jnp.float32), pltpu.VMEM((CH, H2), jnp.float32),
          pltpu.VMEM((CH, H2), jnp.float32), pltpu.VMEM((CH, H2), jnp.float32),
          pltpu.VMEM((CH, H), jnp.float32), pltpu.VMEM((CH, H), jnp.float32),
          pltpu.SemaphoreType.DMA, pltpu.SemaphoreType.DMA,
          pltpu.SemaphoreType.DMA, pltpu.SemaphoreType.DMA,
      ],
  )
  return f(g_tab, ab_tab, c_tab, src1, dst1)


def _sc_gather_edges1(g_tab, src1):
  def body(g_tab, src1, og, i_s0, i_s1, bg0, bg1, sg0, sg1, sw0, sw1):
    _gather_pipeline(
        [(g_tab, H2, False)], src1, None, (og,),
        (i_s0, i_s1), (None, None), ((bg0, bg1),),
        (sg0, sg1), (sw0, sw1))

  f = pl.kernel(
      body,
      out_type=jax.ShapeDtypeStruct((E, H2), jnp.float32),
      mesh=_mesh(),
      scratch_types=[
          pltpu.VMEM((CH,), jnp.int32), pltpu.VMEM((CH,), jnp.int32),
          pltpu.VMEM((CH, H2), jnp.float32), pltpu.VMEM((CH, H2), jnp.float32),
          pltpu.SemaphoreType.DMA, pltpu.SemaphoreType.DMA,
          pltpu.SemaphoreType.DMA, pltpu.SemaphoreType.DMA,
      ],
  )
  return f(g_tab, src1)


def _tc_prep_body(emb_ref, wz1_ref, wh1_ref, wr_ref, bz_ref, bh_ref, bur_ref,
                  ab_ref, c_ref):
  emb = emb_ref[...]
  ea = jnp.dot(emb, wz1_ref[...], preferred_element_type=jnp.float32) + bz_ref[...]
  eb = jnp.dot(emb, wh1_ref[...], preferred_element_type=jnp.float32) + bh_ref[...]
  ab_ref[...] = jnp.concatenate([ea, eb], axis=1)
  c_ref[...] = jnp.dot(emb, wr_ref[...],
                       preferred_element_type=jnp.float32) + bur_ref[...]


def _tc_prep(emb, Wz1, Wh1, Wr, bz, bh, bur):
  return pl.pallas_call(
      _tc_prep_body,
      out_shape=[jax.ShapeDtypeStruct((V, H2), jnp.float32),
                 jax.ShapeDtypeStruct((V, H), jnp.float32)],
  )(emb, Wz1, Wh1, Wr, bz, bh, bur)


EB = 512


def _pair_swap(x):
  nxt = pltpu.roll(x, x.shape[0] - 1, 0)
  prv = pltpu.roll(x, 1, 0)
  row = lax.broadcasted_iota(jnp.int32, x.shape, 0)
  return jnp.where((row & 1) == 0, nxt, prv)


def _tc_gru_body(m_ref, rm_ref, g_ref, ab_ref, pr_ref,
                 wz2_ref, wh2_ref, ur_ref, mo_ref, rmo_ref):
  g = g_ref[...]
  ab = ab_ref[...]
  s = g[:, :H] - _pair_swap(m_ref[...])
  arm = g[:, H:] - _pair_swap(rm_ref[...])
  z = jax.nn.sigmoid(
      ab[:, :H] + jnp.dot(s, wz2_ref[...],
                          preferred_element_type=jnp.float32))
  mt = jnp.tanh(
      ab[:, H:] + jnp.dot(arm, wh2_ref[...],
                          preferred_element_type=jnp.float32))
  mo = (1.0 - z) * s + z * mt
  r = jax.nn.sigmoid(
      pr_ref[...] + jnp.dot(mo, ur_ref[...],
                            preferred_element_type=jnp.float32))
  mo_ref[...] = mo
  rmo_ref[...] = r * mo


def _tc_gru(m, rm, g, ab, pr, Wz2, Wh2, Ur):
  eb_spec = pl.BlockSpec((EB, H), lambda i: (i, 0))
  e2_spec = pl.BlockSpec((EB, H2), lambda i: (i, 0))
  w_spec = pl.BlockSpec((H, H), lambda i: (0, 0))
  return pl.pallas_call(
      _tc_gru_body,
      grid=(E // EB,),
      in_specs=[eb_spec, eb_spec, e2_spec, e2_spec, eb_spec] + [w_spec] * 3,
      out_specs=[eb_spec] * 2,
      out_shape=[jax.ShapeDtypeStruct((E, H), jnp.float32)] * 2,
  )(m, rm, g, ab, pr, Wz2, Wh2, Ur)


NB = 400
VP = 1024


def _tc_final_body(x_ref, h0_ref, h1_ref, tv_ref, widf_ref, pt_ref,
                   wl1_ref, wl2_ref, blr, wo_ref, bo_ref,
                   uu1_ref, uu2_ref, uu3_ref, bur2, us_ref, bsr,
                   acc_ref):
  pid = pl.program_id(0)

  @pl.when(pid == 0)
  def _():
    for i in range(4):
      acc_ref[i] = 0.0

  h = h0_ref[...] + h1_ref[...]
  x = x_ref[...]
  tv = tv_ref[...]

  qp = jax.nn.relu(
      jnp.dot(h, wl1_ref[...], preferred_element_type=jnp.float32)
      + jnp.dot(tv, wl2_ref[...], preferred_element_type=jnp.float32)
      + blr[...])
  q = jnp.dot(qp, wo_ref[...], preferred_element_type=jnp.float32) + bo_ref[...]

  pp = jax.nn.relu(
      jnp.dot(x, uu1_ref[...], preferred_element_type=jnp.float32)
      + jnp.dot(h, uu2_ref[...], preferred_element_type=jnp.float32)
      + jnp.dot(tv, uu3_ref[...], preferred_element_type=jnp.float32)
      + bur2[...])
  p = jnp.dot(pp, us_ref[...], preferred_element_type=jnp.float32) + bsr[...]

  pt = pt_ref[...]
  p_loss = jnp.sum(jnp.maximum(p, 0.0) - p * pt
                   + jnp.log(1.0 + jnp.exp(-jnp.abs(p))))
  p_hit = jnp.sum(jnp.where((p > 0.0) == (pt > 0.5), 1.0, 0.0))

  widf = widf_ref[...]
  coli = lax.broadcasted_iota(jnp.int32, q.shape, 1)
  col = coli.astype(jnp.float32)
  onehot = jnp.where(col == widf, 1.0, 0.0)
  q_sel = jnp.sum(q * onehot, axis=1, keepdims=True)
  q_max = jnp.max(q, axis=1, keepdims=True)
  lse = jnp.log(jnp.sum(jnp.exp(q - q_max), axis=1, keepdims=True)) + q_max
  q_loss = jnp.sum(lse - q_sel)

  am = jnp.min(jnp.where(q == q_max, coli, VP), axis=1, keepdims=True)
  q_hit = jnp.sum(jnp.where(am.astype(jnp.float32) == widf, 1.0, 0.0))

  acc_ref[0] += q_loss
  acc_ref[1] += p_loss
  acc_ref[2] += q_hit
  acc_ref[3] += p_hit


def _tc_final(x, h2, tv, widf, pt, Wl, bl, Wo_p, bo_p, Uu, bu, Us, bs):
  nb_spec = pl.BlockSpec((NB, H), lambda i: (i, 0))
  n1_spec = pl.BlockSpec((NB, 1), lambda i: (i, 0))
  full = lambda shape: pl.BlockSpec(shape, lambda i: tuple(0 for _ in shape))
  return pl.pallas_call(
      _tc_final_body,
      grid=(N // NB,),
      in_specs=[nb_spec, nb_spec, nb_spec, nb_spec, n1_spec, n1_spec,
                full((H, H)), full((H, H)), full((1, H)),
                full((H, VP)), full((1, VP)),
                full((H, H)), full((H, H)), full((H, H)), full((1, H)),
                full((H, 1)), full((1, 1))],
      out_specs=pl.BlockSpec(memory_space=pltpu.MemorySpace.SMEM),
      out_shape=jax.ShapeDtypeStruct((4,), jnp.float32),
  )(x, h2[0], h2[1], tv, widf, pt,
    Wl[:H], Wl[H:], bl.reshape(1, H), Wo_p, bo_p,
    Uu[:H], Uu[H:2 * H], Uu[2 * H:], bu.reshape(1, H), Us, bs.reshape(1, 1))


def kernel(wid, edge_index, tree_id, tree_vec, m0, rm0, p_targets, emb,
           Wz, bz, Wh, bh, Wr, Ur, bur, Wl, bl, Wo, bo, Uu, bu, Us, bs):
  src = edge_index[0].astype(jnp.int32)
  dst = edge_index[1].astype(jnp.int32)
  dst3 = dst.reshape(E // (MEG * CH), MEG, CH)
  wid_p = jnp.concatenate(
      [wid.astype(jnp.int32), jnp.zeros((N_PAD - N,), jnp.int32)])
  tid_p = jnp.concatenate(
      [tree_id.astype(jnp.int32), jnp.zeros((N_PAD - N,), jnp.int32)])
  zeros_n = jnp.zeros((NA, H), jnp.float32)

  ab2, c2 = _tc_prep(emb, Wz[:H], Wh[:H], Wr,
                     bz.reshape(1, H), bh.reshape(1, H), bur.reshape(1, H))

  x_p, ab_p, c_p, tv_p = _sc_gather_nodes(emb, ab2, c2, tree_vec, wid_p, tid_p)

  g1 = _sc_scatter2(m0, rm0, dst3, zeros_n)
  gg1, ab_e, pr_e = _sc_gather_edges3(g1, ab_p, c_p, src, dst)
  m1, rm1 = _tc_gru(m0, rm0, gg1, ab_e, pr_e, Wz[H:], Wh[H:], Ur)

  g2 = _sc_scatter2(m1, rm1, dst3, zeros_n)
  gg2 = _sc_gather_edges1(g2, src)
  m2, _ = _tc_gru(m1, rm1, gg2, ab_e, pr_e, Wz[H:], Wh[H:], Ur)

  h2 = _sc_scatter_half(m2, dst3, zeros_n)

  Wo_p = jnp.concatenate([Wo, jnp.zeros((H, VP - V), jnp.float32)], axis=1)
  bo_p = jnp.concatenate([bo, jnp.full((VP - V,), -1e9, jnp.float32)])
  widf = wid.astype(jnp.float32).reshape(N, 1)
  ptf = p_targets.astype(jnp.float32).reshape(N, 1)

  acc = _tc_final(x_p[:N], h2[:, :N], tv_p[:N], widf, ptf,
                  Wl, bl, Wo_p, bo_p.reshape(1, VP), Uu, bu, Us, bs)

  n_trees = float(T)
  q_loss = acc[0] / n_trees
  p_loss = acc[1] / n_trees
  q_acc = acc[2] / float(N)
  p_acc = acc[3] / float(N)
  return q_loss, p_loss, q_acc, p_acc


# speedup vs baseline: 2.5967x; 1.0187x over previous
import jax
import jax.numpy as jnp
from jax import lax
from jax.experimental import pallas as pl
from jax.experimental.pallas import tpu as pltpu
from jax.experimental.pallas import tpu_sc as plsc

N = 10000
E = 320000
H = 128
H2 = 2 * H
L = 128
V = 1000
T = 256

NC = 2
NS = 16
NW = NC * NS
CH = 80
MEG = 2

N_PAD = 10240
NA = 10112

_mesh = lambda: plsc.VectorSubcoreMesh(
    core_axis_name="c", subcore_axis_name="s", num_cores=NC, num_subcores=NS)


def _worker_id():
  return lax.axis_index("s") * NC + lax.axis_index("c")


def _drain(dummy_src, dst, sem):
  pltpu.make_async_copy(dummy_src, dst, sem).wait()


def _scatter_accumulate(tab, dst3, acc, tile_mg0, n_meg, idx2, rows, sems):
  sem_l, sem_a = sems

  def load(mi, b):
    mg = tile_mg0 + mi
    pltpu.async_copy(dst3.at[mg], idx2[b], sem_l[b])
    pltpu.async_copy(tab.at[pl.ds(mg * MEG * CH, MEG * CH)], rows[b], sem_l[b])

  def wait_load(b):
    _drain(dst3.at[0], idx2[b], sem_l[b])
    _drain(tab.at[pl.ds(0, MEG * CH)], rows[b], sem_l[b])

  def fire_adds(b):
    for j in range(MEG):
      pltpu.async_copy(rows[b].at[pl.ds(j * CH, CH)], acc.at[idx2[b].at[j]],
                       sem_a[b], add=True)

  def wait_adds(b):
    _drain(tab.at[pl.ds(0, MEG * CH)], rows[b], sem_a[b])

  load(0, 0)

  def step(mi, _):
    for b in (0, 1):
      m = 2 * mi + b

      @pl.when(m < n_meg)
      def _():
        wait_load(b)
        fire_adds(b)

        @pl.when(m + 1 < n_meg)
        def _():
          load(m + 1, 1 - b)

        wait_adds(b)
    return ()

  lax.fori_loop(0, (n_meg + 1) // 2, step, (), unroll=False)


def _sc_scatter2_body(m, rm, dst3, zeros, g, idx2a, idx2b, rowsa, rowsb, acc,
                      sl0, sl1, sa0, sa1):
  c = lax.axis_index("c")
  s = lax.axis_index("s")
  rows_per_tile = NA // NS
  r0 = s * rows_per_tile
  pltpu.sync_copy(zeros.at[pl.ds(r0, rows_per_tile)],
                  acc.at[pl.ds(r0, rows_per_tile)])
  plsc.subcore_barrier()

  per_tile_meg = (E // NS) // (MEG * CH)
  sems = ((sl0, sl1), (sa0, sa1))

  @pl.when(c == 0)
  def _():
    _scatter_accumulate(m, dst3, acc, s * per_tile_meg, per_tile_meg,
                        (idx2a, idx2b), (rowsa, rowsb), sems)

  @pl.when(c == 1)
  def _():
    _scatter_accumulate(rm, dst3, acc, s * per_tile_meg, per_tile_meg,
                        (idx2a, idx2b), (rowsa, rowsb), sems)

  plsc.subcore_barrier()

  @pl.when(c == 0)
  def _():
    pltpu.sync_copy(acc.at[pl.ds(r0, rows_per_tile)],
                    g.at[pl.ds(r0, rows_per_tile), pl.ds(0, H)])

  @pl.when(c == 1)
  def _():
    pltpu.sync_copy(acc.at[pl.ds(r0, rows_per_tile)],
                    g.at[pl.ds(r0, rows_per_tile), pl.ds(H, H)])


def _sc_scatter2(m, rm, dst3, zeros):
  f = pl.kernel(
      _sc_scatter2_body,
      out_type=jax.ShapeDtypeStruct((NA, H2), jnp.float32),
      mesh=_mesh(),
      scratch_types=[
          pltpu.VMEM((MEG, CH), jnp.int32),
          pltpu.VMEM((MEG, CH), jnp.int32),
          pltpu.VMEM((MEG * CH, H), jnp.float32),
          pltpu.VMEM((MEG * CH, H), jnp.float32),
          pltpu.VMEM_SHARED((NA, H), jnp.float32),
          pltpu.SemaphoreType.DMA,
          pltpu.SemaphoreType.DMA,
          pltpu.SemaphoreType.DMA,
          pltpu.SemaphoreType.DMA,
      ],
  )
  return f(m, rm, dst3, zeros)


def _sc_scatter_half_body(m, dst3, zeros, hp, idx2a, idx2b, rowsa, rowsb, acc,
                          sl0, sl1, sa0, sa1):
  c = lax.axis_index("c")
  s = lax.axis_index("s")
  rows_per_tile = NA // NS
  r0 = s * rows_per_tile
  pltpu.sync_copy(zeros.at[pl.ds(r0, rows_per_tile)],
                  acc.at[pl.ds(r0, rows_per_tile)])
  plsc.subcore_barrier()

  sems = ((sl0, sl1), (sa0, sa1))

  @pl.when(c == 0)
  def _():
    _scatter_accumulate(m, dst3, acc, s * 64, 64,
                        (idx2a, idx2b), (rowsa, rowsb), sems)

  @pl.when(c == 1)
  def _():
    _scatter_accumulate(m, dst3, acc, 1024 + s * 61, 61,
                        (idx2a, idx2b), (rowsa, rowsb), sems)

  plsc.subcore_barrier()
  pltpu.sync_copy(acc.at[pl.ds(r0, rows_per_tile)],
                  hp.at[c, pl.ds(r0, rows_per_tile)])


def _sc_scatter_half(m, dst3, zeros):
  f = pl.kernel(
      _sc_scatter_half_body,
      out_type=jax.ShapeDtypeStruct((NC, NA, H), jnp.float32),
      mesh=_mesh(),
      scratch_types=[
          pltpu.VMEM((MEG, CH), jnp.int32),
          pltpu.VMEM((MEG, CH), jnp.int32),
          pltpu.VMEM((MEG * CH, H), jnp.float32),
          pltpu.VMEM((MEG * CH, H), jnp.float32),
          pltpu.VMEM_SHARED((NA, H), jnp.float32),
          pltpu.SemaphoreType.DMA,
          pltpu.SemaphoreType.DMA,
          pltpu.SemaphoreType.DMA,
          pltpu.SemaphoreType.DMA,
      ],
  )
  return f(m, dst3, zeros)


def _gather_pipeline(streams, src1, dst1, outs, idx_s, idx_d, bufs, sem_g,
                     sem_w):
  w = _worker_id()
  per_w = E // NW
  n_ch = per_w // CH
  any_dst = any(use_d for _, _, use_d in streams)

  def fire_gathers(ch, b):
    base = w * per_w + ch * CH
    pltpu.sync_copy(src1.at[pl.ds(base, CH)], idx_s[b])
    if any_dst:
      pltpu.sync_copy(dst1.at[pl.ds(base, CH)], idx_d[b])
    for k, (tab, _, use_d) in enumerate(streams):
      idx = idx_d[b] if use_d else idx_s[b]
      pltpu.async_copy(tab.at[idx], bufs[k][b], sem_g[b])

  def wait_gathers(b):
    for k, (tab, _, _) in enumerate(streams):
      _drain(tab.at[pl.ds(0, CH)], bufs[k][b], sem_g[b])

  def fire_wb(ch, b):
    base = w * per_w + ch * CH
    for k in range(len(streams)):
      pltpu.async_copy(bufs[k][b], outs[k].at[pl.ds(base, CH)], sem_w[b])

  def wait_wb(b):
    for k, (tab, _, _) in enumerate(streams):
      _drain(tab.at[pl.ds(0, CH)], bufs[k][b], sem_w[b])

  fire_gathers(0, 0)

  def step(i, _):
    for b in (0, 1):
      ch = 2 * i + b
      wait_gathers(b)
      fire_wb(ch, b)

      @pl.when(ch >= 1)
      def _():
        wait_wb(1 - b)

      @pl.when(ch + 1 < n_ch)
      def _():
        fire_gathers(ch + 1, 1 - b)
    return ()

  lax.fori_loop(0, n_ch // 2, step, (), unroll=False)
  b_last = (n_ch - 1) % 2
  wait_gathers(b_last)
  fire_wb(n_ch - 1, b_last)
  wait_wb(1 - b_last)
  wait_wb(b_last)


def _sc_gather_edges3(g_tab, ab_tab, c_tab, src1, dst1):
  def body(g_tab, ab_tab, c_tab, src1, dst1, og, oab, oc,
           i_s0, i_s1, i_d0, i_d1, bg0, bg1, bab0, bab1, bc0, bc1,
           sg0, sg1, sw0, sw1):
    _gather_pipeline(
        [(g_tab, H2, False), (ab_tab, H2, False), (c_tab, H, True)],
        src1, dst1, (og, oab, oc),
        (i_s0, i_s1), (i_d0, i_d1),
        ((bg0, bg1), (bab0, bab1), (bc0, bc1)),
        (sg0, sg1), (sw0, sw1))

  f = pl.kernel(
      body,
      out_type=[jax.ShapeDtypeStruct((E, H2), jnp.float32),
                jax.ShapeDtypeStruct((E, H2), jnp.float32),
                jax.ShapeDtypeStruct((E, H), jnp.float32)],
      mesh=_mesh(),
      scratch_types=[
          pltpu.VMEM((CH,), jnp.int32), pltpu.VMEM((CH,), jnp.int32),
          pltpu.VMEM((CH,), jnp.int32), pltpu.VMEM((CH,), jnp.int32),
          pltpu.VMEM((CH, H2), jnp.float32), pltpu.VMEM((CH, H2), jnp.float32),
          pltpu.VMEM((CH, H2), jnp.float32), pltpu.VMEM((CH, H2), jnp.float32),
          pltpu.VMEM((CH, H), jnp.float32), pltpu.VMEM((CH, H), jnp.float32),
          pltpu.SemaphoreType.DMA, pltpu.SemaphoreType.DMA,
          pltpu.SemaphoreType.DMA, pltpu.SemaphoreType.DMA,
      ],
  )
  return f(g_tab, ab_tab, c_tab, src1, dst1)


def _sc_gather_edges1(g_tab, src1):
  def body(g_tab, src1, og, i_s0, i_s1, bg0, bg1, sg0, sg1, sw0, sw1):
    _gather_pipeline(
        [(g_tab, H2, False)], src1, None, (og,),
        (i_s0, i_s1), (None, None), ((bg0, bg1),),
        (sg0, sg1), (sw0, sw1))

  f = pl.kernel(
      body,
      out_type=jax.ShapeDtypeStruct((E, H2), jnp.float32),
      mesh=_mesh(),
      scratch_types=[
          pltpu.VMEM((CH,), jnp.int32), pltpu.VMEM((CH,), jnp.int32),
          pltpu.VMEM((CH, H2), jnp.float32), pltpu.VMEM((CH, H2), jnp.float32),
          pltpu.SemaphoreType.DMA, pltpu.SemaphoreType.DMA,
          pltpu.SemaphoreType.DMA, pltpu.SemaphoreType.DMA,
      ],
  )
  return f(g_tab, src1)


NB = 400
VP = 1024


def _tc_tables_body(widf_ref, tidf_ref, emb_ref, tvec_ref,
                    wz1_ref, wh1_ref, wr_ref, bz_ref, bh_ref, bur_ref,
                    x_ref, tv_ref, ab_ref, c_ref):
  colv = lax.broadcasted_iota(jnp.int32, (NB, VP), 1).astype(jnp.float32)
  ow = jnp.where(colv == widf_ref[...], 1.0, 0.0)
  colt = lax.broadcasted_iota(jnp.int32, (NB, T), 1).astype(jnp.float32)
  ot = jnp.where(colt == tidf_ref[...], 1.0, 0.0)
  x = jnp.dot(ow, emb_ref[...], preferred_element_type=jnp.float32)
  tv = jnp.dot(ot, tvec_ref[...], preferred_element_type=jnp.float32)
  a = jnp.dot(x, wz1_ref[...], preferred_element_type=jnp.float32) + bz_ref[...]
  b = jnp.dot(x, wh1_ref[...], preferred_element_type=jnp.float32) + bh_ref[...]
  c = jnp.dot(x, wr_ref[...], preferred_element_type=jnp.float32) + bur_ref[...]
  x_ref[...] = x
  tv_ref[...] = tv
  ab_ref[...] = jnp.concatenate([a, b], axis=1)
  c_ref[...] = c


def _tc_tables(widf, tidf, emb_p, tvec, Wz1, Wh1, Wr, bz, bh, bur):
  nb_spec = pl.BlockSpec((NB, H), lambda i: (i, 0))
  n1_spec = pl.BlockSpec((NB, 1), lambda i: (i, 0))
  full = lambda shape: pl.BlockSpec(shape, lambda i: tuple(0 for _ in shape))
  return pl.pallas_call(
      _tc_tables_body,
      grid=(N // NB,),
      in_specs=[n1_spec, n1_spec, full((VP, H)), full((T, H)),
                full((H, H)), full((H, H)), full((H, H)),
                full((1, H)), full((1, H)), full((1, H))],
      out_specs=[nb_spec, nb_spec, pl.BlockSpec((NB, H2), lambda i: (i, 0)),
                 nb_spec],
      out_shape=[jax.ShapeDtypeStruct((N, H), jnp.float32),
                 jax.ShapeDtypeStruct((N, H), jnp.float32),
                 jax.ShapeDtypeStruct((N, H2), jnp.float32),
                 jax.ShapeDtypeStruct((N, H), jnp.float32)],
  )(widf, tidf, emb_p, tvec, Wz1, Wh1, Wr, bz, bh, bur)


EB = 512


def _pair_swap(x):
  nxt = pltpu.roll(x, x.shape[0] - 1, 0)
  prv = pltpu.roll(x, 1, 0)
  row = lax.broadcasted_iota(jnp.int32, x.shape, 0)
  return jnp.where((row & 1) == 0, nxt, prv)


def _tc_gru_body(m_ref, rm_ref, g_ref, ab_ref, pr_ref,
                 wz2_ref, wh2_ref, ur_ref, mo_ref, rmo_ref):
  g = g_ref[...]
  ab = ab_ref[...]
  a = ab[:, :H]
  b = ab[:, H:]
  s = g[:, :H] - _pair_swap(m_ref[...])
  arm = g[:, H:] - _pair_swap(rm_ref[...])
  z = jax.nn.sigmoid(
      a + jnp.dot(s, wz2_ref[...],
                  preferred_element_type=jnp.float32))
  mt = jnp.tanh(
      b + jnp.dot(arm, wh2_ref[...],
                  preferred_element_type=jnp.float32))
  mo = (1.0 - z) * s + z * mt
  r = jax.nn.sigmoid(
      pr_ref[...] + jnp.dot(mo, ur_ref[...],
                            preferred_element_type=jnp.float32))
  mo_ref[...] = mo
  rmo_ref[...] = r * mo


def _tc_gru(m, rm, g, ab, pr, Wz2, Wh2, Ur):
  eb_spec = pl.BlockSpec((EB, H), lambda i: (i, 0))
  e2_spec = pl.BlockSpec((EB, H2), lambda i: (i, 0))
  w_spec = pl.BlockSpec((H, H), lambda i: (0, 0))
  return pl.pallas_call(
      _tc_gru_body,
      grid=(E // EB,),
      in_specs=[eb_spec, eb_spec, e2_spec, e2_spec, eb_spec] + [w_spec] * 3,
      out_specs=[eb_spec] * 2,
      out_shape=[jax.ShapeDtypeStruct((E, H), jnp.float32)] * 2,
  )(m, rm, g, ab, pr, Wz2, Wh2, Ur)


def _tc_final_body(x_ref, h0_ref, h1_ref, tv_ref, widf_ref, pt_ref,
                   wl1_ref, wl2_ref, blr, wo_ref, bo_ref,
                   uu1_ref, uu2_ref, uu3_ref, bur2, us_ref, bsr,
                   acc_ref):
  pid = pl.program_id(0)

  @pl.when(pid == 0)
  def _():
    for i in range(4):
      acc_ref[i] = 0.0

  h = h0_ref[...] + h1_ref[...]
  x = x_ref[...]
  tv = tv_ref[...]

  qp = jax.nn.relu(
      jnp.dot(h, wl1_ref[...], preferred_element_type=jnp.float32)
      + jnp.dot(tv, wl2_ref[...], preferred_element_type=jnp.float32)
      + blr[...])
  q = jnp.dot(qp, wo_ref[...], preferred_element_type=jnp.float32) + bo_ref[...]

  pp = jax.nn.relu(
      jnp.dot(x, uu1_ref[...], preferred_element_type=jnp.float32)
      + jnp.dot(h, uu2_ref[...], preferred_element_type=jnp.float32)
      + jnp.dot(tv, uu3_ref[...], preferred_element_type=jnp.float32)
      + bur2[...])
  p = jnp.dot(pp, us_ref[...], preferred_element_type=jnp.float32) + bsr[...]

  pt = pt_ref[...]
  p_loss = jnp.sum(jnp.maximum(p, 0.0) - p * pt
                   + jnp.log(1.0 + jnp.exp(-jnp.abs(p))))
  p_hit = jnp.sum(jnp.where((p > 0.0) == (pt > 0.5), 1.0, 0.0))

  widf = widf_ref[...]
  coli = lax.broadcasted_iota(jnp.int32, q.shape, 1)
  col = coli.astype(jnp.float32)
  onehot = jnp.where(col == widf, 1.0, 0.0)
  q_sel = jnp.sum(q * onehot, axis=1, keepdims=True)
  q_max = jnp.max(q, axis=1, keepdims=True)
  lse = jnp.log(jnp.sum(jnp.exp(q - q_max), axis=1, keepdims=True)) + q_max
  q_loss = jnp.sum(lse - q_sel)

  am = jnp.min(jnp.where(q == q_max, coli, VP), axis=1, keepdims=True)
  q_hit = jnp.sum(jnp.where(am.astype(jnp.float32) == widf, 1.0, 0.0))

  acc_ref[0] += q_loss
  acc_ref[1] += p_loss
  acc_ref[2] += q_hit
  acc_ref[3] += p_hit


def _tc_final(x, h2, tv, widf, pt, Wl, bl, Wo_p, bo_p, Uu, bu, Us, bs):
  nb_spec = pl.BlockSpec((NB, H), lambda i: (i, 0))
  n1_spec = pl.BlockSpec((NB, 1), lambda i: (i, 0))
  full = lambda shape: pl.BlockSpec(shape, lambda i: tuple(0 for _ in shape))
  return pl.pallas_call(
      _tc_final_body,
      grid=(N // NB,),
      in_specs=[nb_spec, nb_spec, nb_spec, nb_spec, n1_spec, n1_spec,
                full((H, H)), full((H, H)), full((1, H)),
                full((H, VP)), full((1, VP)),
                full((H, H)), full((H, H)), full((H, H)), full((1, H)),
                full((H, 1)), full((1, 1))],
      out_specs=pl.BlockSpec(memory_space=pltpu.MemorySpace.SMEM),
      out_shape=jax.ShapeDtypeStruct((4,), jnp.float32),
  )(x, h2[0], h2[1], tv, widf, pt,
    Wl[:H], Wl[H:], bl.reshape(1, H), Wo_p, bo_p,
    Uu[:H], Uu[H:2 * H], Uu[2 * H:], bu.reshape(1, H), Us, bs.reshape(1, 1))


def kernel(wid, edge_index, tree_id, tree_vec, m0, rm0, p_targets, emb,
           Wz, bz, Wh, bh, Wr, Ur, bur, Wl, bl, Wo, bo, Uu, bu, Us, bs):
  src = edge_index[0].astype(jnp.int32)
  dst = edge_index[1].astype(jnp.int32)
  dst3 = dst.reshape(E // (MEG * CH), MEG, CH)
  zeros_n = jnp.zeros((NA, H), jnp.float32)
  widf = wid.astype(jnp.float32).reshape(N, 1)
  tidf = tree_id.astype(jnp.float32).reshape(N, 1)
  emb_p = jnp.concatenate([emb, jnp.zeros((VP - V, H), jnp.float32)])

  x_n, tv_n, ab_n, c_n = _tc_tables(
      widf, tidf, emb_p, tree_vec, Wz[:H], Wh[:H], Wr,
      bz.reshape(1, H), bh.reshape(1, H), bur.reshape(1, H))

  g1 = _sc_scatter2(m0, rm0, dst3, zeros_n)
  gg1, ab_e, pr_e = _sc_gather_edges3(g1, ab_n, c_n, src, dst)
  m1, rm1 = _tc_gru(m0, rm0, gg1, ab_e, pr_e, Wz[H:], Wh[H:], Ur)

  g2 = _sc_scatter2(m1, rm1, dst3, zeros_n)
  gg2 = _sc_gather_edges1(g2, src)
  m2, _ = _tc_gru(m1, rm1, gg2, ab_e, pr_e, Wz[H:], Wh[H:], Ur)

  h2 = _sc_scatter_half(m2, dst3, zeros_n)

  Wo_p = jnp.concatenate([Wo, jnp.zeros((H, VP - V), jnp.float32)], axis=1)
  bo_p = jnp.concatenate([bo, jnp.full((VP - V,), -1e9, jnp.float32)])
  ptf = p_targets.astype(jnp.float32).reshape(N, 1)

  acc = _tc_final(x_n, h2[:, :N], tv_n, widf, ptf,
                  Wl, bl, Wo_p, bo_p.reshape(1, VP), Uu, bu, Us, bs)

  n_trees = float(T)
  q_loss = acc[0] / n_trees
  p_loss = acc[1] / n_trees
  q_acc = acc[2] / float(N)
  p_acc = acc[3] / float(N)
  return q_loss, p_loss, q_acc, p_acc


# speedup vs baseline: 2.6900x; 1.0360x over previous
import jax
import jax.numpy as jnp
from jax import lax
from jax.experimental import pallas as pl
from jax.experimental.pallas import tpu as pltpu
from jax.experimental.pallas import tpu_sc as plsc

N = 10000
E = 320000
H = 128
H2 = 2 * H
L = 128
V = 1000
T = 256

NC = 2
NS = 16
NW = NC * NS
CH = 80
MEG = 2

N_PAD = 10240
NA = 10112

_mesh = lambda: plsc.VectorSubcoreMesh(
    core_axis_name="c", subcore_axis_name="s", num_cores=NC, num_subcores=NS)


def _worker_id():
  return lax.axis_index("s") * NC + lax.axis_index("c")


def _drain(dummy_src, dst, sem):
  pltpu.make_async_copy(dummy_src, dst, sem).wait()


def _scatter_accumulate(tab, dst3, acc, tile_mg0, n_meg, idx2, rows, sems):
  sem_l, sem_a = sems

  def load(mi, b):
    mg = tile_mg0 + mi
    pltpu.async_copy(dst3.at[mg], idx2[b], sem_l[b])
    pltpu.async_copy(tab.at[pl.ds(mg * MEG * CH, MEG * CH)], rows[b], sem_l[b])

  def wait_load(b):
    _drain(dst3.at[0], idx2[b], sem_l[b])
    _drain(tab.at[pl.ds(0, MEG * CH)], rows[b], sem_l[b])

  def fire_adds(b):
    for j in range(MEG):
      pltpu.async_copy(rows[b].at[pl.ds(j * CH, CH)], acc.at[idx2[b].at[j]],
                       sem_a[b], add=True)

  def wait_adds(b):
    _drain(tab.at[pl.ds(0, MEG * CH)], rows[b], sem_a[b])

  load(0, 0)

  def step(mi, _):
    for b in (0, 1):
      m = 2 * mi + b

      @pl.when(m < n_meg)
      def _():
        wait_load(b)
        fire_adds(b)

        @pl.when(m + 1 < n_meg)
        def _():
          load(m + 1, 1 - b)

        wait_adds(b)
    return ()

  lax.fori_loop(0, (n_meg + 1) // 2, step, (), unroll=False)


def _sc_scatter2_body(m, rm, dst3, zeros, g, idx2a, idx2b, rowsa, rowsb, acc,
                      sl0, sl1, sa0, sa1):
  c = lax.axis_index("c")
  s = lax.axis_index("s")
  rows_per_tile = NA // NS
  r0 = s * rows_per_tile
  pltpu.sync_copy(zeros.at[pl.ds(r0, rows_per_tile)],
                  acc.at[pl.ds(r0, rows_per_tile)])
  plsc.subcore_barrier()

  per_tile_meg = (E // NS) // (MEG * CH)
  sems = ((sl0, sl1), (sa0, sa1))

  @pl.when(c == 0)
  def _():
    _scatter_accumulate(m, dst3, acc, s * per_tile_meg, per_tile_meg,
                        (idx2a, idx2b), (rowsa, rowsb), sems)

  @pl.when(c == 1)
  def _():
    _scatter_accumulate(rm, dst3, acc, s * per_tile_meg, per_tile_meg,
                        (idx2a, idx2b), (rowsa, rowsb), sems)

  plsc.subcore_barrier()

  @pl.when(c == 0)
  def _():
    pltpu.sync_copy(acc.at[pl.ds(r0, rows_per_tile)],
                    g.at[pl.ds(r0, rows_per_tile), pl.ds(0, H)])

  @pl.when(c == 1)
  def _():
    pltpu.sync_copy(acc.at[pl.ds(r0, rows_per_tile)],
                    g.at[pl.ds(r0, rows_per_tile), pl.ds(H, H)])


def _sc_scatter2(m, rm, dst3, zeros):
  f = pl.kernel(
      _sc_scatter2_body,
      out_type=jax.ShapeDtypeStruct((NA, H2), jnp.float32),
      mesh=_mesh(),
      scratch_types=[
          pltpu.VMEM((MEG, CH), jnp.int32),
          pltpu.VMEM((MEG, CH), jnp.int32),
          pltpu.VMEM((MEG * CH, H), jnp.float32),
          pltpu.VMEM((MEG * CH, H), jnp.float32),
          pltpu.VMEM_SHARED((NA, H), jnp.float32),
          pltpu.SemaphoreType.DMA,
          pltpu.SemaphoreType.DMA,
          pltpu.SemaphoreType.DMA,
          pltpu.SemaphoreType.DMA,
      ],
  )
  return f(m, rm, dst3, zeros)


def _sc_scatter_half_body(m, dst3, zeros, hp, idx2a, idx2b, rowsa, rowsb, acc,
                          sl0, sl1, sa0, sa1):
  c = lax.axis_index("c")
  s = lax.axis_index("s")
  rows_per_tile = NA // NS
  r0 = s * rows_per_tile
  pltpu.sync_copy(zeros.at[pl.ds(r0, rows_per_tile)],
                  acc.at[pl.ds(r0, rows_per_tile)])
  plsc.subcore_barrier()

  sems = ((sl0, sl1), (sa0, sa1))

  @pl.when(c == 0)
  def _():
    _scatter_accumulate(m, dst3, acc, s * 64, 64,
                        (idx2a, idx2b), (rowsa, rowsb), sems)

  @pl.when(c == 1)
  def _():
    _scatter_accumulate(m, dst3, acc, 1024 + s * 61, 61,
                        (idx2a, idx2b), (rowsa, rowsb), sems)

  plsc.subcore_barrier()
  pltpu.sync_copy(acc.at[pl.ds(r0, rows_per_tile)],
                  hp.at[c, pl.ds(r0, rows_per_tile)])


def _sc_scatter_half(m, dst3, zeros):
  f = pl.kernel(
      _sc_scatter_half_body,
      out_type=jax.ShapeDtypeStruct((NC, NA, H), jnp.float32),
      mesh=_mesh(),
      scratch_types=[
          pltpu.VMEM((MEG, CH), jnp.int32),
          pltpu.VMEM((MEG, CH), jnp.int32),
          pltpu.VMEM((MEG * CH, H), jnp.float32),
          pltpu.VMEM((MEG * CH, H), jnp.float32),
          pltpu.VMEM_SHARED((NA, H), jnp.float32),
          pltpu.SemaphoreType.DMA,
          pltpu.SemaphoreType.DMA,
          pltpu.SemaphoreType.DMA,
          pltpu.SemaphoreType.DMA,
      ],
  )
  return f(m, dst3, zeros)


def _gather_pipeline(streams, src1, dst1, outs, idx_s, idx_d, bufs, sem_g,
                     sem_w):
  w = _worker_id()
  per_w = E // NW
  n_ch = per_w // CH
  any_dst = any(use_d for _, _, use_d in streams)

  def fire_gathers(ch, b):
    base = w * per_w + ch * CH
    pltpu.sync_copy(src1.at[pl.ds(base, CH)], idx_s[b])
    if any_dst:
      pltpu.sync_copy(dst1.at[pl.ds(base, CH)], idx_d[b])
    for k, (tab, _, use_d) in enumerate(streams):
      idx = idx_d[b] if use_d else idx_s[b]
      pltpu.async_copy(tab.at[idx], bufs[k][b], sem_g[b])

  def wait_gathers(b):
    for k, (tab, _, _) in enumerate(streams):
      _drain(tab.at[pl.ds(0, CH)], bufs[k][b], sem_g[b])

  def fire_wb(ch, b):
    base = w * per_w + ch * CH
    for k in range(len(streams)):
      pltpu.async_copy(bufs[k][b], outs[k].at[pl.ds(base, CH)], sem_w[b])

  def wait_wb(b):
    for k, (tab, _, _) in enumerate(streams):
      _drain(tab.at[pl.ds(0, CH)], bufs[k][b], sem_w[b])

  fire_gathers(0, 0)

  def step(i, _):
    for b in (0, 1):
      ch = 2 * i + b
      wait_gathers(b)
      fire_wb(ch, b)

      @pl.when(ch >= 1)
      def _():
        wait_wb(1 - b)

      @pl.when(ch + 1 < n_ch)
      def _():
        fire_gathers(ch + 1, 1 - b)
    return ()

  lax.fori_loop(0, n_ch // 2, step, (), unroll=False)
  b_last = (n_ch - 1) % 2
  wait_gathers(b_last)
  fire_wb(n_ch - 1, b_last)
  wait_wb(1 - b_last)
  wait_wb(b_last)


def _sc_gather_edges3(g_tab, ab_tab, c_tab, src1, dst1):
  def body(g_tab, ab_tab, c_tab, src1, dst1, og, oab, oc,
           i_s0, i_s1, i_d0, i_d1, bg0, bg1, bab0, bab1, bc0, bc1,
           sg0, sg1, sw0, sw1):
    _gather_pipeline(
        [(g_tab, H2, False), (ab_tab, H2, False), (c_tab, H, True)],
        src1, dst1, (og, oab, oc),
        (i_s0, i_s1), (i_d0, i_d1),
        ((bg0, bg1), (bab0, bab1), (bc0, bc1)),
        (sg0, sg1), (sw0, sw1))

  f = pl.kernel(
      body,
      out_type=[jax.ShapeDtypeStruct((E, H2), jnp.float32),
                jax.ShapeDtypeStruct((E, H2), jnp.float32),
                jax.ShapeDtypeStruct((E, H), jnp.float32)],
      mesh=_mesh(),
      scratch_types=[
          pltpu.VMEM((CH,), jnp.int32), pltpu.VMEM((CH,), jnp.int32),
          pltpu.VMEM((CH,), jnp.int32), pltpu.VMEM((CH,), jnp.int32),
          pltpu.VMEM((CH, H2), jnp.float32), pltpu.VMEM((CH, H2), jnp.float32),
          pltpu.VMEM((CH, H2), jnp.float32), pltpu.VMEM((CH, H2), jnp.float32),
          pltpu.VMEM((CH, H), jnp.float32), pltpu.VMEM((CH, H), jnp.float32),
          pltpu.SemaphoreType.DMA, pltpu.SemaphoreType.DMA,
          pltpu.SemaphoreType.DMA, pltpu.SemaphoreType.DMA,
      ],
  )
  return f(g_tab, ab_tab, c_tab, src1, dst1)


def _sc_gather_edges1(g_tab, src1):
  def body(g_tab, src1, og, i_s0, i_s1, bg0, bg1, sg0, sg1, sw0, sw1):
    _gather_pipeline(
        [(g_tab, H2, False)], src1, None, (og,),
        (i_s0, i_s1), (None, None), ((bg0, bg1),),
        (sg0, sg1), (sw0, sw1))

  f = pl.kernel(
      body,
      out_type=jax.ShapeDtypeStruct((E, H2), jnp.float32),
      mesh=_mesh(),
      scratch_types=[
          pltpu.VMEM((CH,), jnp.int32), pltpu.VMEM((CH,), jnp.int32),
          pltpu.VMEM((CH, H2), jnp.float32), pltpu.VMEM((CH, H2), jnp.float32),
          pltpu.SemaphoreType.DMA, pltpu.SemaphoreType.DMA,
          pltpu.SemaphoreType.DMA, pltpu.SemaphoreType.DMA,
      ],
  )
  return f(g_tab, src1)


NB = 400
VP = 1024


def _tc_tables_body(widf_ref, tidf_ref, emb_ref, tvec_ref,
                    wz1_ref, wh1_ref, wr_ref, bz_ref, bh_ref, bur_ref,
                    x_ref, tv_ref, ab_ref, c_ref):
  colv = lax.broadcasted_iota(jnp.int32, (NB, VP), 1).astype(jnp.float32)
  ow = jnp.where(colv == widf_ref[...], 1.0, 0.0)
  colt = lax.broadcasted_iota(jnp.int32, (NB, T), 1).astype(jnp.float32)
  ot = jnp.where(colt == tidf_ref[...], 1.0, 0.0)
  x = jnp.dot(ow, emb_ref[...], preferred_element_type=jnp.float32)
  tv = jnp.dot(ot, tvec_ref[...], preferred_element_type=jnp.float32)
  a = jnp.dot(x, wz1_ref[...], preferred_element_type=jnp.float32) + bz_ref[...]
  b = jnp.dot(x, wh1_ref[...], preferred_element_type=jnp.float32) + bh_ref[...]
  c = jnp.dot(x, wr_ref[...], preferred_element_type=jnp.float32) + bur_ref[...]
  x_ref[...] = x
  tv_ref[...] = tv
  ab_ref[...] = jnp.concatenate([a, b], axis=1)
  c_ref[...] = c


def _tc_tables(widf, tidf, emb_p, tvec, Wz1, Wh1, Wr, bz, bh, bur):
  nb_spec = pl.BlockSpec((NB, H), lambda i: (i, 0))
  n1_spec = pl.BlockSpec((NB, 1), lambda i: (i, 0))
  full = lambda shape: pl.BlockSpec(shape, lambda i: tuple(0 for _ in shape))
  return pl.pallas_call(
      _tc_tables_body,
      grid=(N // NB,),
      in_specs=[n1_spec, n1_spec, full((VP, H)), full((T, H)),
                full((H, H)), full((H, H)), full((H, H)),
                full((1, H)), full((1, H)), full((1, H))],
      out_specs=[nb_spec, nb_spec, pl.BlockSpec((NB, H2), lambda i: (i, 0)),
                 nb_spec],
      out_shape=[jax.ShapeDtypeStruct((N, H), jnp.float32),
                 jax.ShapeDtypeStruct((N, H), jnp.float32),
                 jax.ShapeDtypeStruct((N, H2), jnp.float32),
                 jax.ShapeDtypeStruct((N, H), jnp.float32)],
  )(widf, tidf, emb_p, tvec, Wz1, Wh1, Wr, bz, bh, bur)


EB = 512


def _pair_swap(x):
  nxt = pltpu.roll(x, x.shape[0] - 1, 0)
  prv = pltpu.roll(x, 1, 0)
  row = lax.broadcasted_iota(jnp.int32, x.shape, 0)
  return jnp.where((row & 1) == 0, nxt, prv)


def _tc_gru_body(m_ref, rm_ref, g_ref, ab_ref, pr_ref,
                 wz2_ref, wh2_ref, ur_ref, mo_ref, rmo_ref):
  g = g_ref[...]
  ab = ab_ref[...]
  a = ab[:, :H]
  b = ab[:, H:]
  s = g[:, :H] - _pair_swap(m_ref[...])
  arm = g[:, H:] - _pair_swap(rm_ref[...])
  z = jax.nn.sigmoid(
      a + jnp.dot(s, wz2_ref[...],
                  preferred_element_type=jnp.float32))
  mt = jnp.tanh(
      b + jnp.dot(arm, wh2_ref[...],
                  preferred_element_type=jnp.float32))
  mo = (1.0 - z) * s + z * mt
  r = jax.nn.sigmoid(
      pr_ref[...] + jnp.dot(mo, ur_ref[...],
                            preferred_element_type=jnp.float32))
  mo_ref[...] = mo
  rmo_ref[...] = r * mo


def _tc_gru2_body(m_ref, rm_ref, g_ref, ab_ref, wz2_ref, wh2_ref, mo_ref):
  g = g_ref[...]
  ab = ab_ref[...]
  s = g[:, :H] - _pair_swap(m_ref[...])
  arm = g[:, H:] - _pair_swap(rm_ref[...])
  z = jax.nn.sigmoid(
      ab[:, :H] + jnp.dot(s, wz2_ref[...],
                          preferred_element_type=jnp.float32))
  mt = jnp.tanh(
      ab[:, H:] + jnp.dot(arm, wh2_ref[...],
                          preferred_element_type=jnp.float32))
  mo_ref[...] = (1.0 - z) * s + z * mt


def _tc_gru2(m, rm, g, ab, Wz2, Wh2):
  eb_spec = pl.BlockSpec((EB, H), lambda i: (i, 0))
  e2_spec = pl.BlockSpec((EB, H2), lambda i: (i, 0))
  w_spec = pl.BlockSpec((H, H), lambda i: (0, 0))
  return pl.pallas_call(
      _tc_gru2_body,
      grid=(E // EB,),
      in_specs=[eb_spec, eb_spec, e2_spec, e2_spec] + [w_spec] * 2,
      out_specs=eb_spec,
      out_shape=jax.ShapeDtypeStruct((E, H), jnp.float32),
  )(m, rm, g, ab, Wz2, Wh2)


def _tc_gru(m, rm, g, ab, pr, Wz2, Wh2, Ur):
  eb_spec = pl.BlockSpec((EB, H), lambda i: (i, 0))
  e2_spec = pl.BlockSpec((EB, H2), lambda i: (i, 0))
  w_spec = pl.BlockSpec((H, H), lambda i: (0, 0))
  return pl.pallas_call(
      _tc_gru_body,
      grid=(E // EB,),
      in_specs=[eb_spec, eb_spec, e2_spec, e2_spec, eb_spec] + [w_spec] * 3,
      out_specs=[eb_spec] * 2,
      out_shape=[jax.ShapeDtypeStruct((E, H), jnp.float32)] * 2,
  )(m, rm, g, ab, pr, Wz2, Wh2, Ur)


def _tc_final_body(x_ref, h0_ref, h1_ref, tv_ref, widf_ref, pt_ref,
                   wl1_ref, wl2_ref, blr, wo_ref, bo_ref,
                   uu1_ref, uu2_ref, uu3_ref, bur2, us_ref, bsr,
                   acc_ref):
  pid = pl.program_id(0)

  @pl.when(pid == 0)
  def _():
    for i in range(4):
      acc_ref[i] = 0.0

  h = h0_ref[...] + h1_ref[...]
  x = x_ref[...]
  tv = tv_ref[...]

  qp = jax.nn.relu(
      jnp.dot(h, wl1_ref[...], preferred_element_type=jnp.float32)
      + jnp.dot(tv, wl2_ref[...], preferred_element_type=jnp.float32)
      + blr[...])
  q = jnp.dot(qp, wo_ref[...], preferred_element_type=jnp.float32) + bo_ref[...]

  pp = jax.nn.relu(
      jnp.dot(x, uu1_ref[...], preferred_element_type=jnp.float32)
      + jnp.dot(h, uu2_ref[...], preferred_element_type=jnp.float32)
      + jnp.dot(tv, uu3_ref[...], preferred_element_type=jnp.float32)
      + bur2[...])
  p = jnp.dot(pp, us_ref[...], preferred_element_type=jnp.float32) + bsr[...]

  pt = pt_ref[...]
  p_loss = jnp.sum(jnp.maximum(p, 0.0) - p * pt
                   + jnp.log(1.0 + jnp.exp(-jnp.abs(p))))
  p_hit = jnp.sum(jnp.where((p > 0.0) == (pt > 0.5), 1.0, 0.0))

  widf = widf_ref[...]
  coli = lax.broadcasted_iota(jnp.int32, q.shape, 1)
  col = coli.astype(jnp.float32)
  onehot = jnp.where(col == widf, 1.0, 0.0)
  q_sel = jnp.sum(q * onehot, axis=1, keepdims=True)
  q_max = jnp.max(q, axis=1, keepdims=True)
  lse = jnp.log(jnp.sum(jnp.exp(q - q_max), axis=1, keepdims=True)) + q_max
  q_loss = jnp.sum(lse - q_sel)

  am = jnp.min(jnp.where(q == q_max, coli, VP), axis=1, keepdims=True)
  q_hit = jnp.sum(jnp.where(am.astype(jnp.float32) == widf, 1.0, 0.0))

  acc_ref[0] += q_loss
  acc_ref[1] += p_loss
  acc_ref[2] += q_hit
  acc_ref[3] += p_hit


def _tc_final(x, h2, tv, widf, pt, Wl, bl, Wo_p, bo_p, Uu, bu, Us, bs):
  nb_spec = pl.BlockSpec((NB, H), lambda i: (i, 0))
  n1_spec = pl.BlockSpec((NB, 1), lambda i: (i, 0))
  full = lambda shape: pl.BlockSpec(shape, lambda i: tuple(0 for _ in shape))
  return pl.pallas_call(
      _tc_final_body,
      grid=(N // NB,),
      in_specs=[nb_spec, nb_spec, nb_spec, nb_spec, n1_spec, n1_spec,
                full((H, H)), full((H, H)), full((1, H)),
                full((H, VP)), full((1, VP)),
                full((H, H)), full((H, H)), full((H, H)), full((1, H)),
                full((H, 1)), full((1, 1))],
      out_specs=pl.BlockSpec(memory_space=pltpu.MemorySpace.SMEM),
      out_shape=jax.ShapeDtypeStruct((4,), jnp.float32),
  )(x, h2[0], h2[1], tv, widf, pt,
    Wl[:H], Wl[H:], bl.reshape(1, H), Wo_p, bo_p,
    Uu[:H], Uu[H:2 * H], Uu[2 * H:], bu.reshape(1, H), Us, bs.reshape(1, 1))


def kernel(wid, edge_index, tree_id, tree_vec, m0, rm0, p_targets, emb,
           Wz, bz, Wh, bh, Wr, Ur, bur, Wl, bl, Wo, bo, Uu, bu, Us, bs):
  src = edge_index[0].astype(jnp.int32)
  dst = edge_index[1].astype(jnp.int32)
  dst3 = dst.reshape(E // (MEG * CH), MEG, CH)
  zeros_n = jnp.zeros((NA, H), jnp.float32)
  widf = wid.astype(jnp.float32).reshape(N, 1)
  tidf = tree_id.astype(jnp.float32).reshape(N, 1)
  emb_p = jnp.concatenate([emb, jnp.zeros((VP - V, H), jnp.float32)])

  x_n, tv_n, ab_n, c_n = _tc_tables(
      widf, tidf, emb_p, tree_vec, Wz[:H], Wh[:H], Wr,
      bz.reshape(1, H), bh.reshape(1, H), bur.reshape(1, H))

  g1 = _sc_scatter2(m0, rm0, dst3, zeros_n)
  gg1, ab_e, pr_e = _sc_gather_edges3(g1, ab_n, c_n, src, dst)
  m1, rm1 = _tc_gru(m0, rm0, gg1, ab_e, pr_e, Wz[H:], Wh[H:], Ur)

  g2 = _sc_scatter2(m1, rm1, dst3, zeros_n)
  gg2 = _sc_gather_edges1(g2, src)
  m2 = _tc_gru2(m1, rm1, gg2, ab_e, Wz[H:], Wh[H:])

  h2 = _sc_scatter_half(m2, dst3, zeros_n)

  Wo_p = jnp.concatenate([Wo, jnp.zeros((H, VP - V), jnp.float32)], axis=1)
  bo_p = jnp.concatenate([bo, jnp.full((VP - V,), -1e9, jnp.float32)])
  ptf = p_targets.astype(jnp.float32).reshape(N, 1)

  acc = _tc_final(x_n, h2[:, :N], tv_n, widf, ptf,
                  Wl, bl, Wo_p, bo_p.reshape(1, VP), Uu, bu, Us, bs)

  n_trees = float(T)
  q_loss = acc[0] / n_trees
  p_loss = acc[1] / n_trees
  q_acc = acc[2] / float(N)
  p_acc = acc[3] / float(N)
  return q_loss, p_loss, q_acc, p_acc


# speedup vs baseline: 2.7569x; 1.0249x over previous
import jax
import jax.numpy as jnp
from jax import lax
from jax.experimental import pallas as pl
from jax.experimental.pallas import tpu as pltpu
from jax.experimental.pallas import tpu_sc as plsc

N = 10000
E = 320000
H = 128
H2 = 2 * H
L = 128
V = 1000
T = 256

NC = 2
NS = 16
NW = NC * NS
CH = 80
MEG = 2

NA = 10112

EA = 192000
EBH = 128000
MG_A = EA // (MEG * CH)

_mesh = lambda: plsc.VectorSubcoreMesh(
    core_axis_name="c", subcore_axis_name="s", num_cores=NC, num_subcores=NS)


def _worker_id():
  return lax.axis_index("s") * NC + lax.axis_index("c")


def _drain(dummy_src, dst, sem):
  pltpu.make_async_copy(dummy_src, dst, sem).wait()


def _scatter_accumulate(tab, dst3, acc, tile_mg0, n_meg, idx2, rows, sems,
                        mg_off=0):
  sem_l, sem_a = sems

  def load(mi, b):
    mg = tile_mg0 + mi
    pltpu.async_copy(dst3.at[mg + mg_off], idx2[b], sem_l[b])
    pltpu.async_copy(tab.at[pl.ds(mg * MEG * CH, MEG * CH)], rows[b], sem_l[b])

  def wait_load(b):
    _drain(dst3.at[0], idx2[b], sem_l[b])
    _drain(tab.at[pl.ds(0, MEG * CH)], rows[b], sem_l[b])

  def fire_adds(b):
    for j in range(MEG):
      pltpu.async_copy(rows[b].at[pl.ds(j * CH, CH)], acc.at[idx2[b].at[j]],
                       sem_a[b], add=True)

  def wait_adds(b):
    _drain(tab.at[pl.ds(0, MEG * CH)], rows[b], sem_a[b])

  load(0, 0)

  def step(mi, _):
    for b in (0, 1):
      m = 2 * mi + b

      @pl.when(m < n_meg)
      def _():
        wait_load(b)
        fire_adds(b)

        @pl.when(m + 1 < n_meg)
        def _():
          load(m + 1, 1 - b)

        wait_adds(b)
    return ()

  lax.fori_loop(0, (n_meg + 1) // 2, step, (), unroll=False)


def _acc_zero_and_barrier(zeros, acc, r0, n_rows):
  pltpu.sync_copy(zeros.at[pl.ds(r0, n_rows)], acc.at[pl.ds(r0, n_rows)])
  plsc.subcore_barrier()


def _sc_scatter2_part1_body(m, rm, dst3, zeros, part, idx2a, idx2b,
                            rowsa, rowsb, acc, sl0, sl1, sa0, sa1):
  c = lax.axis_index("c")
  s = lax.axis_index("s")
  rows_per_tile = NA // NS
  r0 = s * rows_per_tile
  _acc_zero_and_barrier(zeros, acc, r0, rows_per_tile)

  per_tile_meg = MG_A // NS
  sems = ((sl0, sl1), (sa0, sa1))

  @pl.when(c == 0)
  def _():
    _scatter_accumulate(m, dst3, acc, s * per_tile_meg, per_tile_meg,
                        (idx2a, idx2b), (rowsa, rowsb), sems)

  @pl.when(c == 1)
  def _():
    _scatter_accumulate(rm, dst3, acc, s * per_tile_meg, per_tile_meg,
                        (idx2a, idx2b), (rowsa, rowsb), sems)

  plsc.subcore_barrier()
  pltpu.sync_copy(acc.at[pl.ds(r0, rows_per_tile)],
                  part.at[c, pl.ds(r0, rows_per_tile)])


def _sc_scatter2_part1(mA, rmA, dst3, zeros):
  f = pl.kernel(
      _sc_scatter2_part1_body,
      out_type=jax.ShapeDtypeStruct((NC, NA, H), jnp.float32),
      mesh=_mesh(),
      scratch_types=[
          pltpu.VMEM((MEG, CH), jnp.int32),
          pltpu.VMEM((MEG, CH), jnp.int32),
          pltpu.VMEM((MEG * CH, H), jnp.float32),
          pltpu.VMEM((MEG * CH, H), jnp.float32),
          pltpu.VMEM_SHARED((NA, H), jnp.float32),
          pltpu.SemaphoreType.DMA,
          pltpu.SemaphoreType.DMA,
          pltpu.SemaphoreType.DMA,
          pltpu.SemaphoreType.DMA,
      ],
  )
  return f(mA, rmA, dst3, zeros)


def _sc_scatter2_part2_body(m, rm, dst3, part, g, idx2a, idx2b,
                            rowsa, rowsb, acc, sl0, sl1, sa0, sa1):
  c = lax.axis_index("c")
  s = lax.axis_index("s")
  rows_per_tile = NA // NS
  r0 = s * rows_per_tile
  pltpu.sync_copy(part.at[c, pl.ds(r0, rows_per_tile)],
                  acc.at[pl.ds(r0, rows_per_tile)])
  plsc.subcore_barrier()

  per_tile_meg = (EBH // (MEG * CH)) // NS
  sems = ((sl0, sl1), (sa0, sa1))

  @pl.when(c == 0)
  def _():
    _scatter_accumulate(m, dst3, acc, s * per_tile_meg, per_tile_meg,
                        (idx2a, idx2b), (rowsa, rowsb), sems, mg_off=MG_A)

  @pl.when(c == 1)
  def _():
    _scatter_accumulate(rm, dst3, acc, s * per_tile_meg, per_tile_meg,
                        (idx2a, idx2b), (rowsa, rowsb), sems, mg_off=MG_A)

  plsc.subcore_barrier()

  @pl.when(c == 0)
  def _():
    pltpu.sync_copy(acc.at[pl.ds(r0, rows_per_tile)],
                    g.at[pl.ds(r0, rows_per_tile), pl.ds(0, H)])

  @pl.when(c == 1)
  def _():
    pltpu.sync_copy(acc.at[pl.ds(r0, rows_per_tile)],
                    g.at[pl.ds(r0, rows_per_tile), pl.ds(H, H)])


def _sc_scatter2_part2(mB, rmB, dst3, part):
  f = pl.kernel(
      _sc_scatter2_part2_body,
      out_type=jax.ShapeDtypeStruct((NA, H2), jnp.float32),
      mesh=_mesh(),
      scratch_types=[
          pltpu.VMEM((MEG, CH), jnp.int32),
          pltpu.VMEM((MEG, CH), jnp.int32),
          pltpu.VMEM((MEG * CH, H), jnp.float32),
          pltpu.VMEM((MEG * CH, H), jnp.float32),
          pltpu.VMEM_SHARED((NA, H), jnp.float32),
          pltpu.SemaphoreType.DMA,
          pltpu.SemaphoreType.DMA,
          pltpu.SemaphoreType.DMA,
          pltpu.SemaphoreType.DMA,
      ],
  )
  return f(mB, rmB, dst3, part)


def _sc_scatter_half_body(mA, mB, dst3, zeros, hp, idx2a, idx2b, rowsa, rowsb,
                          acc, sl0, sl1, sa0, sa1):
  c = lax.axis_index("c")
  s = lax.axis_index("s")
  rows_per_tile = NA // NS
  r0 = s * rows_per_tile
  _acc_zero_and_barrier(zeros, acc, r0, rows_per_tile)

  sems = ((sl0, sl1), (sa0, sa1))

  @pl.when(c == 0)
  def _():
    _scatter_accumulate(mA, dst3, acc, s * (MG_A // NS), MG_A // NS,
                        (idx2a, idx2b), (rowsa, rowsb), sems)

  @pl.when(c == 1)
  def _():
    nmeg = (EBH // (MEG * CH)) // NS
    _scatter_accumulate(mB, dst3, acc, s * nmeg, nmeg,
                        (idx2a, idx2b), (rowsa, rowsb), sems, mg_off=MG_A)

  plsc.subcore_barrier()
  pltpu.sync_copy(acc.at[pl.ds(r0, rows_per_tile)],
                  hp.at[c, pl.ds(r0, rows_per_tile)])


def _sc_scatter_half(mA, mB, dst3, zeros):
  f = pl.kernel(
      _sc_scatter_half_body,
      out_type=jax.ShapeDtypeStruct((NC, NA, H), jnp.float32),
      mesh=_mesh(),
      scratch_types=[
          pltpu.VMEM((MEG, CH), jnp.int32),
          pltpu.VMEM((MEG, CH), jnp.int32),
          pltpu.VMEM((MEG * CH, H), jnp.float32),
          pltpu.VMEM((MEG * CH, H), jnp.float32),
          pltpu.VMEM_SHARED((NA, H), jnp.float32),
          pltpu.SemaphoreType.DMA,
          pltpu.SemaphoreType.DMA,
          pltpu.SemaphoreType.DMA,
          pltpu.SemaphoreType.DMA,
      ],
  )
  return f(mA, mB, dst3, zeros)


def _gather_pipeline(streams, src1, dst1, outs, idx_s, idx_d, bufs, sem_g,
                     sem_w, e0, n_e):
  w = _worker_id()
  per_w = n_e // NW
  n_ch = per_w // CH
  any_dst = any(use_d for _, _, use_d in streams)

  def fire_gathers(ch, b):
    base = w * per_w + ch * CH
    pltpu.sync_copy(src1.at[pl.ds(e0 + base, CH)], idx_s[b])
    if any_dst:
      pltpu.sync_copy(dst1.at[pl.ds(e0 + base, CH)], idx_d[b])
    for k, (tab, _, use_d) in enumerate(streams):
      idx = idx_d[b] if use_d else idx_s[b]
      pltpu.async_copy(tab.at[idx], bufs[k][b], sem_g[b])

  def wait_gathers(b):
    for k, (tab, _, _) in enumerate(streams):
      _drain(tab.at[pl.ds(0, CH)], bufs[k][b], sem_g[b])

  def fire_wb(ch, b):
    base = w * per_w + ch * CH
    for k in range(len(streams)):
      pltpu.async_copy(bufs[k][b], outs[k].at[pl.ds(base, CH)], sem_w[b])

  def wait_wb(b):
    for k, (tab, _, _) in enumerate(streams):
      _drain(tab.at[pl.ds(0, CH)], bufs[k][b], sem_w[b])

  fire_gathers(0, 0)

  def step(i, _):
    for b in (0, 1):
      ch = 2 * i + b
      wait_gathers(b)
      fire_wb(ch, b)

      @pl.when(ch >= 1)
      def _():
        wait_wb(1 - b)

      @pl.when(ch + 1 < n_ch)
      def _():
        fire_gathers(ch + 1, 1 - b)
    return ()

  lax.fori_loop(0, n_ch // 2, step, (), unroll=False)
  if n_ch % 2 == 1:
    wait_gathers(0)
    fire_wb(n_ch - 1, 0)
    wait_wb(1)
    wait_wb(0)
  else:
    wait_wb(1)


def _sc_gather_edges3(g_tab, ab_tab, c_tab, src1, dst1, e0, n_e):
  def body(g_tab, ab_tab, c_tab, src1, dst1, og, oab, oc,
           i_s0, i_s1, i_d0, i_d1, bg0, bg1, bab0, bab1, bc0, bc1,
           sg0, sg1, sw0, sw1):
    _gather_pipeline(
        [(g_tab, H2, False), (ab_tab, H2, False), (c_tab, H, True)],
        src1, dst1, (og, oab, oc),
        (i_s0, i_s1), (i_d0, i_d1),
        ((bg0, bg1), (bab0, bab1), (bc0, bc1)),
        (sg0, sg1), (sw0, sw1), e0, n_e)

  f = pl.kernel(
      body,
      out_type=[jax.ShapeDtypeStruct((n_e, H2), jnp.float32),
                jax.ShapeDtypeStruct((n_e, H2), jnp.float32),
                jax.ShapeDtypeStruct((n_e, H), jnp.float32)],
      mesh=_mesh(),
      scratch_types=[
          pltpu.VMEM((CH,), jnp.int32), pltpu.VMEM((CH,), jnp.int32),
          pltpu.VMEM((CH,), jnp.int32), pltpu.VMEM((CH,), jnp.int32),
          pltpu.VMEM((CH, H2), jnp.float32), pltpu.VMEM((CH, H2), jnp.float32),
          pltpu.VMEM((CH, H2), jnp.float32), pltpu.VMEM((CH, H2), jnp.float32),
          pltpu.VMEM((CH, H), jnp.float32), pltpu.VMEM((CH, H), jnp.float32),
          pltpu.SemaphoreType.DMA, pltpu.SemaphoreType.DMA,
          pltpu.SemaphoreType.DMA, pltpu.SemaphoreType.DMA,
      ],
  )
  return f(g_tab, ab_tab, c_tab, src1, dst1)


def _sc_gather_edges1(g_tab, src1, e0, n_e):
  def body(g_tab, src1, og, i_s0, i_s1, bg0, bg1, sg0, sg1, sw0, sw1):
    _gather_pipeline(
        [(g_tab, H2, False)], src1, None, (og,),
        (i_s0, i_s1), (None, None), ((bg0, bg1),),
        (sg0, sg1), (sw0, sw1), e0, n_e)

  f = pl.kernel(
      body,
      out_type=jax.ShapeDtypeStruct((n_e, H2), jnp.float32),
      mesh=_mesh(),
      scratch_types=[
          pltpu.VMEM((CH,), jnp.int32), pltpu.VMEM((CH,), jnp.int32),
          pltpu.VMEM((CH, H2), jnp.float32), pltpu.VMEM((CH, H2), jnp.float32),
          pltpu.SemaphoreType.DMA, pltpu.SemaphoreType.DMA,
          pltpu.SemaphoreType.DMA, pltpu.SemaphoreType.DMA,
      ],
  )
  return f(g_tab, src1)


NB = 400
VP = 1024


def _tc_tables_body(widf_ref, tidf_ref, emb_ref, tvec_ref,
                    wz1_ref, wh1_ref, wr_ref, bz_ref, bh_ref, bur_ref,
                    x_ref, tv_ref, ab_ref, c_ref):
  colv = lax.broadcasted_iota(jnp.int32, (NB, VP), 1).astype(jnp.float32)
  ow = jnp.where(colv == widf_ref[...], 1.0, 0.0)
  colt = lax.broadcasted_iota(jnp.int32, (NB, T), 1).astype(jnp.float32)
  ot = jnp.where(colt == tidf_ref[...], 1.0, 0.0)
  x = jnp.dot(ow, emb_ref[...], preferred_element_type=jnp.float32)
  tv = jnp.dot(ot, tvec_ref[...], preferred_element_type=jnp.float32)
  a = jnp.dot(x, wz1_ref[...], preferred_element_type=jnp.float32) + bz_ref[...]
  b = jnp.dot(x, wh1_ref[...], preferred_element_type=jnp.float32) + bh_ref[...]
  c = jnp.dot(x, wr_ref[...], preferred_element_type=jnp.float32) + bur_ref[...]
  x_ref[...] = x
  tv_ref[...] = tv
  ab_ref[...] = jnp.concatenate([a, b], axis=1)
  c_ref[...] = c


def _tc_tables(widf, tidf, emb_p, tvec, Wz1, Wh1, Wr, bz, bh, bur):
  nb_spec = pl.BlockSpec((NB, H), lambda i: (i, 0))
  n1_spec = pl.BlockSpec((NB, 1), lambda i: (i, 0))
  full = lambda shape: pl.BlockSpec(shape, lambda i: tuple(0 for _ in shape))
  return pl.pallas_call(
      _tc_tables_body,
      grid=(N // NB,),
      in_specs=[n1_spec, n1_spec, full((VP, H)), full((T, H)),
                full((H, H)), full((H, H)), full((H, H)),
                full((1, H)), full((1, H)), full((1, H))],
      out_specs=[nb_spec, nb_spec, pl.BlockSpec((NB, H2), lambda i: (i, 0)),
                 nb_spec],
      out_shape=[jax.ShapeDtypeStruct((N, H), jnp.float32),
                 jax.ShapeDtypeStruct((N, H), jnp.float32),
                 jax.ShapeDtypeStruct((N, H2), jnp.float32),
                 jax.ShapeDtypeStruct((N, H), jnp.float32)],
  )(widf, tidf, emb_p, tvec, Wz1, Wh1, Wr, bz, bh, bur)


EB = 512


def _pair_swap(x):
  nxt = pltpu.roll(x, x.shape[0] - 1, 0)
  prv = pltpu.roll(x, 1, 0)
  row = lax.broadcasted_iota(jnp.int32, x.shape, 0)
  return jnp.where((row & 1) == 0, nxt, prv)


def _tc_gru_body(m_ref, rm_ref, g_ref, ab_ref, pr_ref,
                 wz2_ref, wh2_ref, ur_ref, mo_ref, rmo_ref):
  g = g_ref[...]
  ab = ab_ref[...]
  a = ab[:, :H]
  b = ab[:, H:]
  s = g[:, :H] - _pair_swap(m_ref[...])
  arm = g[:, H:] - _pair_swap(rm_ref[...])
  z = jax.nn.sigmoid(
      a + jnp.dot(s, wz2_ref[...],
                  preferred_element_type=jnp.float32))
  mt = jnp.tanh(
      b + jnp.dot(arm, wh2_ref[...],
                  preferred_element_type=jnp.float32))
  mo = (1.0 - z) * s + z * mt
  r = jax.nn.sigmoid(
      pr_ref[...] + jnp.dot(mo, ur_ref[...],
                            preferred_element_type=jnp.float32))
  mo_ref[...] = mo
  rmo_ref[...] = r * mo


def _tc_gru2_body(m_ref, rm_ref, g_ref, ab_ref, wz2_ref, wh2_ref, mo_ref):
  g = g_ref[...]
  ab = ab_ref[...]
  s = g[:, :H] - _pair_swap(m_ref[...])
  arm = g[:, H:] - _pair_swap(rm_ref[...])
  z = jax.nn.sigmoid(
      ab[:, :H] + jnp.dot(s, wz2_ref[...],
                          preferred_element_type=jnp.float32))
  mt = jnp.tanh(
      ab[:, H:] + jnp.dot(arm, wh2_ref[...],
                          preferred_element_type=jnp.float32))
  mo_ref[...] = (1.0 - z) * s + z * mt


def _tc_gru2(m, rm, g, ab, Wz2, Wh2, n_e):
  eb_spec = pl.BlockSpec((EB, H), lambda i: (i, 0))
  e2_spec = pl.BlockSpec((EB, H2), lambda i: (i, 0))
  w_spec = pl.BlockSpec((H, H), lambda i: (0, 0))
  return pl.pallas_call(
      _tc_gru2_body,
      grid=(n_e // EB,),
      in_specs=[eb_spec, eb_spec, e2_spec, e2_spec] + [w_spec] * 2,
      out_specs=eb_spec,
      out_shape=jax.ShapeDtypeStruct((n_e, H), jnp.float32),
  )(m, rm, g, ab, Wz2, Wh2)


def _tc_gru(m, rm, g, ab, pr, Wz2, Wh2, Ur, n_e):
  eb_spec = pl.BlockSpec((EB, H), lambda i: (i, 0))
  e2_spec = pl.BlockSpec((EB, H2), lambda i: (i, 0))
  w_spec = pl.BlockSpec((H, H), lambda i: (0, 0))
  return pl.pallas_call(
      _tc_gru_body,
      grid=(n_e // EB,),
      in_specs=[eb_spec, eb_spec, e2_spec, e2_spec, eb_spec] + [w_spec] * 3,
      out_specs=[eb_spec] * 2,
      out_shape=[jax.ShapeDtypeStruct((n_e, H), jnp.float32)] * 2,
  )(m, rm, g, ab, pr, Wz2, Wh2, Ur)


def _tc_final_body(x_ref, h0_ref, h1_ref, tv_ref, widf_ref, pt_ref,
                   wl1_ref, wl2_ref, blr, wo_ref, bo_ref,
                   uu1_ref, uu2_ref, uu3_ref, bur2, us_ref, bsr,
                   acc_ref):
  pid = pl.program_id(0)

  @pl.when(pid == 0)
  def _():
    for i in range(4):
      acc_ref[i] = 0.0

  h = h0_ref[...] + h1_ref[...]
  x = x_ref[...]
  tv = tv_ref[...]

  qp = jax.nn.relu(
      jnp.dot(h, wl1_ref[...], preferred_element_type=jnp.float32)
      + jnp.dot(tv, wl2_ref[...], preferred_element_type=jnp.float32)
      + blr[...])
  q = jnp.dot(qp, wo_ref[...], preferred_element_type=jnp.float32) + bo_ref[...]

  pp = jax.nn.relu(
      jnp.dot(x, uu1_ref[...], preferred_element_type=jnp.float32)
      + jnp.dot(h, uu2_ref[...], preferred_element_type=jnp.float32)
      + jnp.dot(tv, uu3_ref[...], preferred_element_type=jnp.float32)
      + bur2[...])
  p = jnp.dot(pp, us_ref[...], preferred_element_type=jnp.float32) + bsr[...]

  pt = pt_ref[...]
  p_loss = jnp.sum(jnp.maximum(p, 0.0) - p * pt
                   + jnp.log(1.0 + jnp.exp(-jnp.abs(p))))
  p_hit = jnp.sum(jnp.where((p > 0.0) == (pt > 0.5), 1.0, 0.0))

  widf = widf_ref[...]
  coli = lax.broadcasted_iota(jnp.int32, q.shape, 1)
  col = coli.astype(jnp.float32)
  onehot = jnp.where(col == widf, 1.0, 0.0)
  q_sel = jnp.sum(q * onehot, axis=1, keepdims=True)
  q_max = jnp.max(q, axis=1, keepdims=True)
  lse = jnp.log(jnp.sum(jnp.exp(q - q_max), axis=1, keepdims=True)) + q_max
  q_loss = jnp.sum(lse - q_sel)

  am = jnp.min(jnp.where(q == q_max, coli, VP), axis=1, keepdims=True)
  q_hit = jnp.sum(jnp.where(am.astype(jnp.float32) == widf, 1.0, 0.0))

  acc_ref[0] += q_loss
  acc_ref[1] += p_loss
  acc_ref[2] += q_hit
  acc_ref[3] += p_hit


def _tc_final(x, h2, tv, widf, pt, Wl, bl, Wo_p, bo_p, Uu, bu, Us, bs):
  nb_spec = pl.BlockSpec((NB, H), lambda i: (i, 0))
  n1_spec = pl.BlockSpec((NB, 1), lambda i: (i, 0))
  full = lambda shape: pl.BlockSpec(shape, lambda i: tuple(0 for _ in shape))
  return pl.pallas_call(
      _tc_final_body,
      grid=(N // NB,),
      in_specs=[nb_spec, nb_spec, nb_spec, nb_spec, n1_spec, n1_spec,
                full((H, H)), full((H, H)), full((1, H)),
                full((H, VP)), full((1, VP)),
                full((H, H)), full((H, H)), full((H, H)), full((1, H)),
                full((H, 1)), full((1, 1))],
      out_specs=pl.BlockSpec(memory_space=pltpu.MemorySpace.SMEM),
      out_shape=jax.ShapeDtypeStruct((4,), jnp.float32),
  )(x, h2[0], h2[1], tv, widf, pt,
    Wl[:H], Wl[H:], bl.reshape(1, H), Wo_p, bo_p,
    Uu[:H], Uu[H:2 * H], Uu[2 * H:], bu.reshape(1, H), Us, bs.reshape(1, 1))


def kernel(wid, edge_index, tree_id, tree_vec, m0, rm0, p_targets, emb,
           Wz, bz, Wh, bh, Wr, Ur, bur, Wl, bl, Wo, bo, Uu, bu, Us, bs):
  src = edge_index[0].astype(jnp.int32)
  dst = edge_index[1].astype(jnp.int32)
  dst3 = dst.reshape(E // (MEG * CH), MEG, CH)
  zeros_n = jnp.zeros((NA, H), jnp.float32)
  widf = wid.astype(jnp.float32).reshape(N, 1)
  tidf = tree_id.astype(jnp.float32).reshape(N, 1)
  emb_p = jnp.concatenate([emb, jnp.zeros((VP - V, H), jnp.float32)])

  x_n, tv_n, ab_n, c_n = _tc_tables(
      widf, tidf, emb_p, tree_vec, Wz[:H], Wh[:H], Wr,
      bz.reshape(1, H), bh.reshape(1, H), bur.reshape(1, H))

  m0A, m0B = m0[:EA], m0[EA:]
  rm0A, rm0B = rm0[:EA], rm0[EA:]

  p1 = _sc_scatter2_part1(m0A, rm0A, dst3, zeros_n)
  g1 = _sc_scatter2_part2(m0B, rm0B, dst3, p1)
  ggA, abA, prA = _sc_gather_edges3(g1, ab_n, c_n, src, dst, 0, EA)
  ggB, abB, prB = _sc_gather_edges3(g1, ab_n, c_n, src, dst, EA, EBH)
  m1A, rm1A = _tc_gru(m0A, rm0A, ggA, abA, prA, Wz[H:], Wh[H:], Ur, EA)
  m1B, rm1B = _tc_gru(m0B, rm0B, ggB, abB, prB, Wz[H:], Wh[H:], Ur, EBH)

  p2 = _sc_scatter2_part1(m1A, rm1A, dst3, zeros_n)
  g2 = _sc_scatter2_part2(m1B, rm1B, dst3, p2)
  gg2A = _sc_gather_edges1(g2, src, 0, EA)
  gg2B = _sc_gather_edges1(g2, src, EA, EBH)
  m2A = _tc_gru2(m1A, rm1A, gg2A, abA, Wz[H:], Wh[H:], EA)
  m2B = _tc_gru2(m1B, rm1B, gg2B, abB, Wz[H:], Wh[H:], EBH)

  h2 = _sc_scatter_half(m2A, m2B, dst3, zeros_n)

  Wo_p = jnp.concatenate([Wo, jnp.zeros((H, VP - V), jnp.float32)], axis=1)
  bo_p = jnp.concatenate([bo, jnp.full((VP - V,), -1e9, jnp.float32)])
  ptf = p_targets.astype(jnp.float32).reshape(N, 1)

  acc = _tc_final(x_n, h2[:, :N], tv_n, widf, ptf,
                  Wl, bl, Wo_p, bo_p.reshape(1, VP), Uu, bu, Us, bs)

  n_trees = float(T)
  q_loss = acc[0] / n_trees
  p_loss = acc[1] / n_trees
  q_acc = acc[2] / float(N)
  p_acc = acc[3] / float(N)
  return q_loss, p_loss, q_acc, p_acc


# speedup vs baseline: 3.3895x; 1.2295x over previous
import jax
import jax.numpy as jnp
from jax import lax
from jax.experimental import pallas as pl
from jax.experimental.pallas import tpu as pltpu
from jax.experimental.pallas import tpu_sc as plsc

N = 10000
E = 320000
H = 128
H2 = 2 * H
L = 128
V = 1000
T = 256

NC = 2
NS = 16
NW = NC * NS
CH = 80
MEG = 2

NA = 10112

EA = 192000
EBH = 128000
MG_A = EA // (MEG * CH)

_mesh = lambda: plsc.VectorSubcoreMesh(
    core_axis_name="c", subcore_axis_name="s", num_cores=NC, num_subcores=NS)


def _worker_id():
  return lax.axis_index("s") * NC + lax.axis_index("c")


def _pack2(a, b):
  au = lax.bitcast_convert_type(a.astype(jnp.bfloat16), jnp.uint16)
  bu = lax.bitcast_convert_type(b.astype(jnp.bfloat16), jnp.uint16)
  word = (bu.astype(jnp.uint32) << 16) | au.astype(jnp.uint32)
  return lax.bitcast_convert_type(word, jnp.int32)


def _unpack2(w):
  wu = lax.bitcast_convert_type(w, jnp.uint32)
  a = lax.bitcast_convert_type(wu << 16, jnp.float32)
  b = lax.bitcast_convert_type(wu & jnp.uint32(0xFFFF0000), jnp.float32)
  return a, b


def _drain(dummy_src, dst, sem):
  pltpu.make_async_copy(dummy_src, dst, sem).wait()


def _scatter_accumulate(tab, dst3, acc, tile_mg0, n_meg, idx2, rows, sems,
                        mg_off=0, tab_mg_off=None):
  if tab_mg_off is None:
    tab_mg_off = 0
  sem_l, sem_a = sems

  def load(mi, b):
    mg = tile_mg0 + mi
    pltpu.async_copy(dst3.at[mg + mg_off], idx2[b], sem_l[b])
    pltpu.async_copy(tab.at[pl.ds((mg + tab_mg_off) * MEG * CH, MEG * CH)],
                     rows[b], sem_l[b])

  def wait_load(b):
    _drain(dst3.at[0], idx2[b], sem_l[b])
    _drain(tab.at[pl.ds(0, MEG * CH)], rows[b], sem_l[b])

  def fire_adds(b):
    for j in range(MEG):
      pltpu.async_copy(rows[b].at[pl.ds(j * CH, CH)], acc.at[idx2[b].at[j]],
                       sem_a[b], add=True)

  def wait_adds(b):
    _drain(tab.at[pl.ds(0, MEG * CH)], rows[b], sem_a[b])

  load(0, 0)

  def step(mi, _):
    for b in (0, 1):
      m = 2 * mi + b

      @pl.when(m < n_meg)
      def _():
        wait_load(b)
        fire_adds(b)

        @pl.when(m + 1 < n_meg)
        def _():
          load(m + 1, 1 - b)

        wait_adds(b)
    return ()

  lax.fori_loop(0, (n_meg + 1) // 2, step, (), unroll=False)


def _acc_zero_and_barrier(zeros, acc, r0, n_rows):
  pltpu.sync_copy(zeros.at[pl.ds(r0, n_rows)], acc.at[pl.ds(r0, n_rows)])
  plsc.subcore_barrier()


def _sc_scatter2_part1_body(m, rm, dst3, zeros, part, idx2a, idx2b,
                            rowsa, rowsb, acc, sl0, sl1, sa0, sa1):
  c = lax.axis_index("c")
  s = lax.axis_index("s")
  rows_per_tile = NA // NS
  r0 = s * rows_per_tile
  _acc_zero_and_barrier(zeros, acc, r0, rows_per_tile)

  per_tile_meg = MG_A // NS
  sems = ((sl0, sl1), (sa0, sa1))

  @pl.when(c == 0)
  def _():
    _scatter_accumulate(m, dst3, acc, s * per_tile_meg, per_tile_meg,
                        (idx2a, idx2b), (rowsa, rowsb), sems)

  @pl.when(c == 1)
  def _():
    _scatter_accumulate(rm, dst3, acc, s * per_tile_meg, per_tile_meg,
                        (idx2a, idx2b), (rowsa, rowsb), sems)

  plsc.subcore_barrier()
  pltpu.sync_copy(acc.at[pl.ds(r0, rows_per_tile)],
                  part.at[c, pl.ds(r0, rows_per_tile)])


def _sc_scatter2_part1(mA, rmA, dst3, zeros):
  f = pl.kernel(
      _sc_scatter2_part1_body,
      out_type=jax.ShapeDtypeStruct((NC, NA, H), jnp.float32),
      mesh=_mesh(),
      scratch_types=[
          pltpu.VMEM((MEG, CH), jnp.int32),
          pltpu.VMEM((MEG, CH), jnp.int32),
          pltpu.VMEM((MEG * CH, H), jnp.float32),
          pltpu.VMEM((MEG * CH, H), jnp.float32),
          pltpu.VMEM_SHARED((NA, H), jnp.float32),
          pltpu.SemaphoreType.DMA,
          pltpu.SemaphoreType.DMA,
          pltpu.SemaphoreType.DMA,
          pltpu.SemaphoreType.DMA,
      ],
  )
  return f(mA, rmA, dst3, zeros)


def _sc_scatter2_part2_body(tab_mg0, m, rm, dst3, part, g, idx2a, idx2b,
                            rowsa, rowsb, acc, sl0, sl1, sa0, sa1):
  c = lax.axis_index("c")
  s = lax.axis_index("s")
  rows_per_tile = NA // NS
  r0 = s * rows_per_tile
  pltpu.sync_copy(part.at[c, pl.ds(r0, rows_per_tile)],
                  acc.at[pl.ds(r0, rows_per_tile)])
  plsc.subcore_barrier()

  per_tile_meg = (EBH // (MEG * CH)) // NS
  sems = ((sl0, sl1), (sa0, sa1))

  @pl.when(c == 0)
  def _():
    _scatter_accumulate(m, dst3, acc, s * per_tile_meg, per_tile_meg,
                        (idx2a, idx2b), (rowsa, rowsb), sems, mg_off=MG_A,
                        tab_mg_off=tab_mg0)

  @pl.when(c == 1)
  def _():
    _scatter_accumulate(rm, dst3, acc, s * per_tile_meg, per_tile_meg,
                        (idx2a, idx2b), (rowsa, rowsb), sems, mg_off=MG_A,
                        tab_mg_off=tab_mg0)

  plsc.subcore_barrier()

  @pl.when(c == 0)
  def _():
    pltpu.sync_copy(acc.at[pl.ds(r0, rows_per_tile)],
                    g.at[pl.ds(r0, rows_per_tile), pl.ds(0, H)])

  @pl.when(c == 1)
  def _():
    pltpu.sync_copy(acc.at[pl.ds(r0, rows_per_tile)],
                    g.at[pl.ds(r0, rows_per_tile), pl.ds(H, H)])


def _sc_scatter2_part2(mB, rmB, dst3, part, tab_mg0=0):
  import functools as _ft
  f = pl.kernel(
      _ft.partial(_sc_scatter2_part2_body, tab_mg0),
      out_type=jax.ShapeDtypeStruct((NA, H2), jnp.float32),
      mesh=_mesh(),
      scratch_types=[
          pltpu.VMEM((MEG, CH), jnp.int32),
          pltpu.VMEM((MEG, CH), jnp.int32),
          pltpu.VMEM((MEG * CH, H), jnp.float32),
          pltpu.VMEM((MEG * CH, H), jnp.float32),
          pltpu.VMEM_SHARED((NA, H), jnp.float32),
          pltpu.SemaphoreType.DMA,
          pltpu.SemaphoreType.DMA,
          pltpu.SemaphoreType.DMA,
          pltpu.SemaphoreType.DMA,
      ],
  )
  return f(mB, rmB, dst3, part)


def _sc_scatter_half_body(mA, mB, dst3, zeros, hp, idx2a, idx2b, rowsa, rowsb,
                          acc, sl0, sl1, sa0, sa1):
  c = lax.axis_index("c")
  s = lax.axis_index("s")
  rows_per_tile = NA // NS
  r0 = s * rows_per_tile
  _acc_zero_and_barrier(zeros, acc, r0, rows_per_tile)

  sems = ((sl0, sl1), (sa0, sa1))

  @pl.when(c == 0)
  def _():
    _scatter_accumulate(mA, dst3, acc, s * (MG_A // NS), MG_A // NS,
                        (idx2a, idx2b), (rowsa, rowsb), sems)

  @pl.when(c == 1)
  def _():
    nmeg = (EBH // (MEG * CH)) // NS
    _scatter_accumulate(mB, dst3, acc, s * nmeg, nmeg,
                        (idx2a, idx2b), (rowsa, rowsb), sems, mg_off=MG_A)

  plsc.subcore_barrier()
  pltpu.sync_copy(acc.at[pl.ds(r0, rows_per_tile)],
                  hp.at[c, pl.ds(r0, rows_per_tile)])


def _sc_scatter_half(mA, mB, dst3, zeros):
  f = pl.kernel(
      _sc_scatter_half_body,
      out_type=jax.ShapeDtypeStruct((NC, NA, H), jnp.float32),
      mesh=_mesh(),
      scratch_types=[
          pltpu.VMEM((MEG, CH), jnp.int32),
          pltpu.VMEM((MEG, CH), jnp.int32),
          pltpu.VMEM((MEG * CH, H), jnp.float32),
          pltpu.VMEM((MEG * CH, H), jnp.float32),
          pltpu.VMEM_SHARED((NA, H), jnp.float32),
          pltpu.SemaphoreType.DMA,
          pltpu.SemaphoreType.DMA,
          pltpu.SemaphoreType.DMA,
          pltpu.SemaphoreType.DMA,
      ],
  )
  return f(mA, mB, dst3, zeros)


def _gather_pipeline(streams, src1, dst1, outs, idx_s, idx_d, bufs, sem_g,
                     sem_w, e0, n_e):
  w = _worker_id()
  per_w = n_e // NW
  n_ch = per_w // CH
  any_dst = any(use_d for _, _, use_d in streams)

  def fire_gathers(ch, b):
    base = w * per_w + ch * CH
    pltpu.sync_copy(src1.at[pl.ds(e0 + base, CH)], idx_s[b])
    if any_dst:
      pltpu.sync_copy(dst1.at[pl.ds(e0 + base, CH)], idx_d[b])
    for k, (tab, _, use_d) in enumerate(streams):
      idx = idx_d[b] if use_d else idx_s[b]
      pltpu.async_copy(tab.at[idx], bufs[k][b], sem_g[b])

  def wait_gathers(b):
    for k, (tab, _, _) in enumerate(streams):
      _drain(tab.at[pl.ds(0, CH)], bufs[k][b], sem_g[b])

  def fire_wb(ch, b):
    base = w * per_w + ch * CH
    for k in range(len(streams)):
      pltpu.async_copy(bufs[k][b], outs[k].at[pl.ds(base, CH)], sem_w[b])

  def wait_wb(b):
    for k, (tab, _, _) in enumerate(streams):
      _drain(tab.at[pl.ds(0, CH)], bufs[k][b], sem_w[b])

  fire_gathers(0, 0)

  def step(i, _):
    for b in (0, 1):
      ch = 2 * i + b
      wait_gathers(b)
      fire_wb(ch, b)

      @pl.when(ch >= 1)
      def _():
        wait_wb(1 - b)

      @pl.when(ch + 1 < n_ch)
      def _():
        fire_gathers(ch + 1, 1 - b)
    return ()

  lax.fori_loop(0, n_ch // 2, step, (), unroll=False)
  if n_ch % 2 == 1:
    wait_gathers(0)
    fire_wb(n_ch - 1, 0)
    wait_wb(1)
    wait_wb(0)
  else:
    wait_wb(1)


def _sc_gather_edges3(g_tab, ab_tab, c_tab, src1, dst1, e0, n_e):
  def body(g_tab, ab_tab, c_tab, src1, dst1, og, oab, oc,
           i_s0, i_s1, i_d0, i_d1, bg0, bg1, bab0, bab1, bc0, bc1,
           sg0, sg1, sw0, sw1):
    _gather_pipeline(
        [(g_tab, H, False), (ab_tab, H, False), (c_tab, H, True)],
        src1, dst1, (og, oab, oc),
        (i_s0, i_s1), (i_d0, i_d1),
        ((bg0, bg1), (bab0, bab1), (bc0, bc1)),
        (sg0, sg1), (sw0, sw1), e0, n_e)

  f = pl.kernel(
      body,
      out_type=[jax.ShapeDtypeStruct((n_e, H), jnp.int32),
                jax.ShapeDtypeStruct((n_e, H), jnp.int32),
                jax.ShapeDtypeStruct((n_e, H), jnp.float32)],
      mesh=_mesh(),
      scratch_types=[
          pltpu.VMEM((CH,), jnp.int32), pltpu.VMEM((CH,), jnp.int32),
          pltpu.VMEM((CH,), jnp.int32), pltpu.VMEM((CH,), jnp.int32),
          pltpu.VMEM((CH, H), jnp.int32), pltpu.VMEM((CH, H), jnp.int32),
          pltpu.VMEM((CH, H), jnp.int32), pltpu.VMEM((CH, H), jnp.int32),
          pltpu.VMEM((CH, H), jnp.float32), pltpu.VMEM((CH, H), jnp.float32),
          pltpu.SemaphoreType.DMA, pltpu.SemaphoreType.DMA,
          pltpu.SemaphoreType.DMA, pltpu.SemaphoreType.DMA,
      ],
  )
  return f(g_tab, ab_tab, c_tab, src1, dst1)


def _sc_gather_edges1(g_tab, src1, e0, n_e):
  def body(g_tab, src1, og, i_s0, i_s1, bg0, bg1, sg0, sg1, sw0, sw1):
    _gather_pipeline(
        [(g_tab, H, False)], src1, None, (og,),
        (i_s0, i_s1), (None, None), ((bg0, bg1),),
        (sg0, sg1), (sw0, sw1), e0, n_e)

  f = pl.kernel(
      body,
      out_type=jax.ShapeDtypeStruct((n_e, H), jnp.int32),
      mesh=_mesh(),
      scratch_types=[
          pltpu.VMEM((CH,), jnp.int32), pltpu.VMEM((CH,), jnp.int32),
          pltpu.VMEM((CH, H), jnp.int32), pltpu.VMEM((CH, H), jnp.int32),
          pltpu.SemaphoreType.DMA, pltpu.SemaphoreType.DMA,
          pltpu.SemaphoreType.DMA, pltpu.SemaphoreType.DMA,
      ],
  )
  return f(g_tab, src1)


NB = 400
VP = 1024


def _tc_tables_body(widf_ref, tidf_ref, emb_ref, tvec_ref,
                    wz1_ref, wh1_ref, wr_ref, bz_ref, bh_ref, bur_ref,
                    x_ref, tv_ref, ab_ref, c_ref):
  colv = lax.broadcasted_iota(jnp.int32, (NB, VP), 1).astype(jnp.float32)
  ow = jnp.where(colv == widf_ref[...], 1.0, 0.0)
  colt = lax.broadcasted_iota(jnp.int32, (NB, T), 1).astype(jnp.float32)
  ot = jnp.where(colt == tidf_ref[...], 1.0, 0.0)
  x = jnp.dot(ow, emb_ref[...], preferred_element_type=jnp.float32)
  tv = jnp.dot(ot, tvec_ref[...], preferred_element_type=jnp.float32)
  a = jnp.dot(x, wz1_ref[...], preferred_element_type=jnp.float32) + bz_ref[...]
  b = jnp.dot(x, wh1_ref[...], preferred_element_type=jnp.float32) + bh_ref[...]
  c = jnp.dot(x, wr_ref[...], preferred_element_type=jnp.float32) + bur_ref[...]
  x_ref[...] = x
  tv_ref[...] = tv
  ab_ref[...] = _pack2(a, b)
  c_ref[...] = c


def _tc_tables(widf, tidf, emb_p, tvec, Wz1, Wh1, Wr, bz, bh, bur):
  nb_spec = pl.BlockSpec((NB, H), lambda i: (i, 0))
  n1_spec = pl.BlockSpec((NB, 1), lambda i: (i, 0))
  full = lambda shape: pl.BlockSpec(shape, lambda i: tuple(0 for _ in shape))
  return pl.pallas_call(
      _tc_tables_body,
      grid=(N // NB,),
      in_specs=[n1_spec, n1_spec, full((VP, H)), full((T, H)),
                full((H, H)), full((H, H)), full((H, H)),
                full((1, H)), full((1, H)), full((1, H))],
      out_specs=[nb_spec, nb_spec, nb_spec, nb_spec],
      out_shape=[jax.ShapeDtypeStruct((N, H), jnp.float32),
                 jax.ShapeDtypeStruct((N, H), jnp.float32),
                 jax.ShapeDtypeStruct((N, H), jnp.int32),
                 jax.ShapeDtypeStruct((N, H), jnp.float32)],
  )(widf, tidf, emb_p, tvec, Wz1, Wh1, Wr, bz, bh, bur)


GBLK = 632


def _tc_pack_g_body(g_ref, o_ref):
  g = g_ref[...]
  o_ref[...] = _pack2(g[:, :H], g[:, H:])


def _tc_pack_g(g):
  return pl.pallas_call(
      _tc_pack_g_body,
      grid=(NA // GBLK,),
      in_specs=[pl.BlockSpec((GBLK, H2), lambda i: (i, 0))],
      out_specs=pl.BlockSpec((GBLK, H), lambda i: (i, 0)),
      out_shape=jax.ShapeDtypeStruct((NA, H), jnp.int32),
  )(g)


EB = 512


def _pair_swap(x):
  nxt = pltpu.roll(x, x.shape[0] - 1, 0)
  prv = pltpu.roll(x, 1, 0)
  row = lax.broadcasted_iota(jnp.int32, x.shape, 0)
  return jnp.where((row & 1) == 0, nxt, prv)


def _tc_gru_body(m_ref, rm_ref, g_ref, ab_ref, pr_ref,
                 wz2_ref, wh2_ref, ur_ref, mo_ref, rmo_ref):
  gnm, gnrm = _unpack2(g_ref[...])
  a, b = _unpack2(ab_ref[...])
  s = gnm - _pair_swap(m_ref[...])
  arm = gnrm - _pair_swap(rm_ref[...])
  z = jax.nn.sigmoid(
      a + jnp.dot(s, wz2_ref[...],
                  preferred_element_type=jnp.float32))
  mt = jnp.tanh(
      b + jnp.dot(arm, wh2_ref[...],
                  preferred_element_type=jnp.float32))
  mo = (1.0 - z) * s + z * mt
  r = jax.nn.sigmoid(
      pr_ref[...] + jnp.dot(mo, ur_ref[...],
                            preferred_element_type=jnp.float32))
  mo_ref[...] = mo
  rmo_ref[...] = r * mo


def _tc_gru2_body(m_ref, rm_ref, g_ref, ab_ref, wz2_ref, wh2_ref, mo_ref):
  gnm, gnrm = _unpack2(g_ref[...])
  a, b = _unpack2(ab_ref[...])
  s = gnm - _pair_swap(m_ref[...])
  arm = gnrm - _pair_swap(rm_ref[...])
  z = jax.nn.sigmoid(
      a + jnp.dot(s, wz2_ref[...],
                  preferred_element_type=jnp.float32))
  mt = jnp.tanh(
      b + jnp.dot(arm, wh2_ref[...],
                  preferred_element_type=jnp.float32))
  mo_ref[...] = (1.0 - z) * s + z * mt


def _tc_gru2(m, rm, g, ab, Wz2, Wh2, n_e):
  eb_spec = pl.BlockSpec((EB, H), lambda i: (i, 0))
  w_spec = pl.BlockSpec((H, H), lambda i: (0, 0))
  return pl.pallas_call(
      _tc_gru2_body,
      grid=(n_e // EB,),
      in_specs=[eb_spec, eb_spec, eb_spec, eb_spec] + [w_spec] * 2,
      out_specs=eb_spec,
      out_shape=jax.ShapeDtypeStruct((n_e, H), jnp.float32),
  )(m, rm, g, ab, Wz2, Wh2)


def _tc_gru(m, rm, g, ab, pr, Wz2, Wh2, Ur, n_e, mrm_blk0=0):
  eb_spec = pl.BlockSpec((EB, H), lambda i: (i, 0))
  mm_spec = pl.BlockSpec((EB, H), lambda i: (mrm_blk0 + i, 0))
  w_spec = pl.BlockSpec((H, H), lambda i: (0, 0))
  return pl.pallas_call(
      _tc_gru_body,
      grid=(n_e // EB,),
      in_specs=[mm_spec, mm_spec, eb_spec, eb_spec, eb_spec] + [w_spec] * 3,
      out_specs=[eb_spec] * 2,
      out_shape=[jax.ShapeDtypeStruct((n_e, H), jnp.float32)] * 2,
  )(m, rm, g, ab, pr, Wz2, Wh2, Ur)


def _tc_final_body(x_ref, h0_ref, h1_ref, tv_ref, widf_ref, pt_ref,
                   wl1_ref, wl2_ref, blr, wo_ref, bo_ref,
                   uu1_ref, uu2_ref, uu3_ref, bur2, us_ref, bsr,
                   acc_ref):
  pid = pl.program_id(0)

  @pl.when(pid == 0)
  def _():
    for i in range(4):
      acc_ref[i] = 0.0

  h = h0_ref[...] + h1_ref[...]
  x = x_ref[...]
  tv = tv_ref[...]

  qp = jax.nn.relu(
      jnp.dot(h, wl1_ref[...], preferred_element_type=jnp.float32)
      + jnp.dot(tv, wl2_ref[...], preferred_element_type=jnp.float32)
      + blr[...])
  q = jnp.dot(qp, wo_ref[...], preferred_element_type=jnp.float32) + bo_ref[...]

  pp = jax.nn.relu(
      jnp.dot(x, uu1_ref[...], preferred_element_type=jnp.float32)
      + jnp.dot(h, uu2_ref[...], preferred_element_type=jnp.float32)
      + jnp.dot(tv, uu3_ref[...], preferred_element_type=jnp.float32)
      + bur2[...])
  p = jnp.dot(pp, us_ref[...], preferred_element_type=jnp.float32) + bsr[...]

  pt = pt_ref[...]
  p_loss = jnp.sum(jnp.maximum(p, 0.0) - p * pt
                   + jnp.log(1.0 + jnp.exp(-jnp.abs(p))))
  p_hit = jnp.sum(jnp.where((p > 0.0) == (pt > 0.5), 1.0, 0.0))

  widf = widf_ref[...]
  coli = lax.broadcasted_iota(jnp.int32, q.shape, 1)
  col = coli.astype(jnp.float32)
  onehot = jnp.where(col == widf, 1.0, 0.0)
  q_sel = jnp.sum(q * onehot, axis=1, keepdims=True)
  q_max = jnp.max(q, axis=1, keepdims=True)
  lse = jnp.log(jnp.sum(jnp.exp(q - q_max), axis=1, keepdims=True)) + q_max
  q_loss = jnp.sum(lse - q_sel)

  am = jnp.min(jnp.where(q == q_max, coli, VP), axis=1, keepdims=True)
  q_hit = jnp.sum(jnp.where(am.astype(jnp.float32) == widf, 1.0, 0.0))

  acc_ref[0] += q_loss
  acc_ref[1] += p_loss
  acc_ref[2] += q_hit
  acc_ref[3] += p_hit


def _tc_final(x, h2, tv, widf, pt, Wl, bl, Wo_p, bo_p, Uu, bu, Us, bs):
  nb_spec = pl.BlockSpec((NB, H), lambda i: (i, 0))
  n1_spec = pl.BlockSpec((NB, 1), lambda i: (i, 0))
  full = lambda shape: pl.BlockSpec(shape, lambda i: tuple(0 for _ in shape))
  return pl.pallas_call(
      _tc_final_body,
      grid=(N // NB,),
      in_specs=[nb_spec, nb_spec, nb_spec, nb_spec, n1_spec, n1_spec,
                full((H, H)), full((H, H)), full((1, H)),
                full((H, VP)), full((1, VP)),
                full((H, H)), full((H, H)), full((H, H)), full((1, H)),
                full((H, 1)), full((1, 1))],
      out_specs=pl.BlockSpec(memory_space=pltpu.MemorySpace.SMEM),
      out_shape=jax.ShapeDtypeStruct((4,), jnp.float32),
  )(x, h2[0], h2[1], tv, widf, pt,
    Wl[:H], Wl[H:], bl.reshape(1, H), Wo_p, bo_p,
    Uu[:H], Uu[H:2 * H], Uu[2 * H:], bu.reshape(1, H), Us, bs.reshape(1, 1))


def kernel(wid, edge_index, tree_id, tree_vec, m0, rm0, p_targets, emb,
           Wz, bz, Wh, bh, Wr, Ur, bur, Wl, bl, Wo, bo, Uu, bu, Us, bs):
  src = edge_index[0].astype(jnp.int32)
  dst = edge_index[1].astype(jnp.int32)
  dst3 = dst.reshape(E // (MEG * CH), MEG, CH)
  zeros_n = jnp.zeros((NA, H), jnp.float32)
  widf = wid.astype(jnp.float32).reshape(N, 1)
  tidf = tree_id.astype(jnp.float32).reshape(N, 1)
  emb_p = jnp.concatenate([emb, jnp.zeros((VP - V, H), jnp.float32)])

  x_n, tv_n, ab_n, c_n = _tc_tables(
      widf, tidf, emb_p, tree_vec, Wz[:H], Wh[:H], Wr,
      bz.reshape(1, H), bh.reshape(1, H), bur.reshape(1, H))

  p1 = _sc_scatter2_part1(m0, rm0, dst3, zeros_n)
  g1 = _sc_scatter2_part2(m0, rm0, dst3, p1, tab_mg0=MG_A)
  g1p = _tc_pack_g(g1)
  ggA, abA, prA = _sc_gather_edges3(g1p, ab_n, c_n, src, dst, 0, EA)
  ggB, abB, prB = _sc_gather_edges3(g1p, ab_n, c_n, src, dst, EA, EBH)
  m1A, rm1A = _tc_gru(m0, rm0, ggA, abA, prA, Wz[H:], Wh[H:], Ur, EA)
  m1B, rm1B = _tc_gru(m0, rm0, ggB, abB, prB, Wz[H:], Wh[H:], Ur, EBH,
                      mrm_blk0=EA // EB)

  p2 = _sc_scatter2_part1(m1A, rm1A, dst3, zeros_n)
  g2 = _sc_scatter2_part2(m1B, rm1B, dst3, p2)
  g2p = _tc_pack_g(g2)
  gg2A = _sc_gather_edges1(g2p, src, 0, EA)
  gg2B = _sc_gather_edges1(g2p, src, EA, EBH)
  m2A = _tc_gru2(m1A, rm1A, gg2A, abA, Wz[H:], Wh[H:], EA)
  m2B = _tc_gru2(m1B, rm1B, gg2B, abB, Wz[H:], Wh[H:], EBH)

  h2 = _sc_scatter_half(m2A, m2B, dst3, zeros_n)

  Wo_p = jnp.concatenate([Wo, jnp.zeros((H, VP - V), jnp.float32)], axis=1)
  bo_p = jnp.concatenate([bo, jnp.full((VP - V,), -1e9, jnp.float32)])
  ptf = p_targets.astype(jnp.float32).reshape(N, 1)

  acc = _tc_final(x_n, h2[:, :N], tv_n, widf, ptf,
                  Wl, bl, Wo_p, bo_p.reshape(1, VP), Uu, bu, Us, bs)

  n_trees = float(T)
  q_loss = acc[0] / n_trees
  p_loss = acc[1] / n_trees
  q_acc = acc[2] / float(N)
  p_acc = acc[3] / float(N)
  return q_loss, p_loss, q_acc, p_acc


# speedup vs baseline: 3.4481x; 1.0173x over previous
import jax
import jax.numpy as jnp
from jax import lax
from jax.experimental import pallas as pl
from jax.experimental.pallas import tpu as pltpu
from jax.experimental.pallas import tpu_sc as plsc

N = 10000
E = 320000
H = 128
H2 = 2 * H
HQ = H // 2
L = 128
V = 1000
T = 256

NC = 2
NS = 16
NW = NC * NS
CH = 80
MEG = 2

NA = 10112

EA = 192000
EBH = 128000
MG_A = EA // (MEG * CH)

_mesh = lambda: plsc.VectorSubcoreMesh(
    core_axis_name="c", subcore_axis_name="s", num_cores=NC, num_subcores=NS)


def _worker_id():
  return lax.axis_index("s") * NC + lax.axis_index("c")


def _pack2(a, b):
  au = lax.bitcast_convert_type(a.astype(jnp.bfloat16), jnp.uint16)
  bu = lax.bitcast_convert_type(b.astype(jnp.bfloat16), jnp.uint16)
  word = (bu.astype(jnp.uint32) << 16) | au.astype(jnp.uint32)
  return lax.bitcast_convert_type(word, jnp.int32)


def _unpack2(w):
  wu = lax.bitcast_convert_type(w, jnp.uint32)
  a = lax.bitcast_convert_type(wu << 16, jnp.float32)
  b = lax.bitcast_convert_type(wu & jnp.uint32(0xFFFF0000), jnp.float32)
  return a, b


def _drain(dummy_src, dst, sem):
  pltpu.make_async_copy(dummy_src, dst, sem).wait()


def _scatter_accumulate(tab, dst3, acc, tile_mg0, n_meg, idx2, rows, sems,
                        mg_off=0, tab_mg_off=None):
  if tab_mg_off is None:
    tab_mg_off = 0
  sem_l, sem_a = sems

  def load(mi, b):
    mg = tile_mg0 + mi
    pltpu.async_copy(dst3.at[mg + mg_off], idx2[b], sem_l[b])
    pltpu.async_copy(tab.at[pl.ds((mg + tab_mg_off) * MEG * CH, MEG * CH)],
                     rows[b], sem_l[b])

  def wait_load(b):
    _drain(dst3.at[0], idx2[b], sem_l[b])
    _drain(tab.at[pl.ds(0, MEG * CH)], rows[b], sem_l[b])

  def fire_adds(b):
    for j in range(MEG):
      pltpu.async_copy(rows[b].at[pl.ds(j * CH, CH)], acc.at[idx2[b].at[j]],
                       sem_a[b], add=True)

  def wait_adds(b):
    _drain(tab.at[pl.ds(0, MEG * CH)], rows[b], sem_a[b])

  load(0, 0)

  def step(mi, _):
    for b in (0, 1):
      m = 2 * mi + b

      @pl.when(m < n_meg)
      def _():
        wait_load(b)
        fire_adds(b)

        @pl.when(m + 1 < n_meg)
        def _():
          load(m + 1, 1 - b)

        wait_adds(b)
    return ()

  lax.fori_loop(0, (n_meg + 1) // 2, step, (), unroll=False)


def _acc_zero_and_barrier(zeros, acc, r0, n_rows):
  pltpu.sync_copy(zeros.at[pl.ds(r0, n_rows)], acc.at[pl.ds(r0, n_rows)])
  plsc.subcore_barrier()


def _sc_scatter2_part1_body(m, rm, dst3, zeros, part, idx2a, idx2b,
                            rowsa, rowsb, acc, sl0, sl1, sa0, sa1):
  c = lax.axis_index("c")
  s = lax.axis_index("s")
  rows_per_tile = NA // NS
  r0 = s * rows_per_tile
  _acc_zero_and_barrier(zeros, acc, r0, rows_per_tile)

  per_tile_meg = MG_A // NS
  sems = ((sl0, sl1), (sa0, sa1))

  @pl.when(c == 0)
  def _():
    _scatter_accumulate(m, dst3, acc, s * per_tile_meg, per_tile_meg,
                        (idx2a, idx2b), (rowsa, rowsb), sems)

  @pl.when(c == 1)
  def _():
    _scatter_accumulate(rm, dst3, acc, s * per_tile_meg, per_tile_meg,
                        (idx2a, idx2b), (rowsa, rowsb), sems)

  plsc.subcore_barrier()
  pltpu.sync_copy(acc.at[pl.ds(r0, rows_per_tile)],
                  part.at[c, pl.ds(r0, rows_per_tile)])


def _sc_scatter2_part1(mA, rmA, dst3, zeros):
  f = pl.kernel(
      _sc_scatter2_part1_body,
      out_type=jax.ShapeDtypeStruct((NC, NA, H), jnp.float32),
      mesh=_mesh(),
      scratch_types=[
          pltpu.VMEM((MEG, CH), jnp.int32),
          pltpu.VMEM((MEG, CH), jnp.int32),
          pltpu.VMEM((MEG * CH, H), jnp.float32),
          pltpu.VMEM((MEG * CH, H), jnp.float32),
          pltpu.VMEM_SHARED((NA, H), jnp.float32),
          pltpu.SemaphoreType.DMA,
          pltpu.SemaphoreType.DMA,
          pltpu.SemaphoreType.DMA,
          pltpu.SemaphoreType.DMA,
      ],
  )
  return f(mA, rmA, dst3, zeros)


def _sc_scatter2_part2_body(tab_mg0, m, rm, dst3, part, g, idx2a, idx2b,
                            rowsa, rowsb, acc, sl0, sl1, sa0, sa1):
  c = lax.axis_index("c")
  s = lax.axis_index("s")
  rows_per_tile = NA // NS
  r0 = s * rows_per_tile
  pltpu.sync_copy(part.at[c, pl.ds(r0, rows_per_tile)],
                  acc.at[pl.ds(r0, rows_per_tile)])
  plsc.subcore_barrier()

  per_tile_meg = (EBH // (MEG * CH)) // NS
  sems = ((sl0, sl1), (sa0, sa1))

  @pl.when(c == 0)
  def _():
    _scatter_accumulate(m, dst3, acc, s * per_tile_meg, per_tile_meg,
                        (idx2a, idx2b), (rowsa, rowsb), sems, mg_off=MG_A,
                        tab_mg_off=tab_mg0)

  @pl.when(c == 1)
  def _():
    _scatter_accumulate(rm, dst3, acc, s * per_tile_meg, per_tile_meg,
                        (idx2a, idx2b), (rowsa, rowsb), sems, mg_off=MG_A,
                        tab_mg_off=tab_mg0)

  plsc.subcore_barrier()

  @pl.when(c == 0)
  def _():
    pltpu.sync_copy(acc.at[pl.ds(r0, rows_per_tile)],
                    g.at[pl.ds(r0, rows_per_tile), pl.ds(0, H)])

  @pl.when(c == 1)
  def _():
    pltpu.sync_copy(acc.at[pl.ds(r0, rows_per_tile)],
                    g.at[pl.ds(r0, rows_per_tile), pl.ds(H, H)])


def _sc_scatter2_part2(mB, rmB, dst3, part, tab_mg0=0):
  import functools as _ft
  f = pl.kernel(
      _ft.partial(_sc_scatter2_part2_body, tab_mg0),
      out_type=jax.ShapeDtypeStruct((NA, H2), jnp.float32),
      mesh=_mesh(),
      scratch_types=[
          pltpu.VMEM((MEG, CH), jnp.int32),
          pltpu.VMEM((MEG, CH), jnp.int32),
          pltpu.VMEM((MEG * CH, H), jnp.float32),
          pltpu.VMEM((MEG * CH, H), jnp.float32),
          pltpu.VMEM_SHARED((NA, H), jnp.float32),
          pltpu.SemaphoreType.DMA,
          pltpu.SemaphoreType.DMA,
          pltpu.SemaphoreType.DMA,
          pltpu.SemaphoreType.DMA,
      ],
  )
  return f(mB, rmB, dst3, part)


def _sc_scatter_h1_body(mA, dst3, zeros, part, idx2a, idx2b, rowsa, rowsb,
                        acc, sl0, sl1, sa0, sa1):
  c = lax.axis_index("c")
  s = lax.axis_index("s")
  rows_per_tile = NA // NS
  r0 = s * rows_per_tile
  _acc_zero_and_barrier(zeros, acc, r0, rows_per_tile)
  sems = ((sl0, sl1), (sa0, sa1))

  @pl.when(c == 0)
  def _():
    _scatter_accumulate(mA, dst3, acc, s * 38, 38,
                        (idx2a, idx2b), (rowsa, rowsb), sems)

  @pl.when(c == 1)
  def _():
    _scatter_accumulate(mA, dst3, acc, 608 + s * 37, 37,
                        (idx2a, idx2b), (rowsa, rowsb), sems)

  plsc.subcore_barrier()
  pltpu.sync_copy(acc.at[pl.ds(r0, rows_per_tile)],
                  part.at[c, pl.ds(r0, rows_per_tile)])


def _sc_scatter_h1(mA, dst3, zeros):
  f = pl.kernel(
      _sc_scatter_h1_body,
      out_type=jax.ShapeDtypeStruct((NC, NA, H), jnp.float32),
      mesh=_mesh(),
      scratch_types=[
          pltpu.VMEM((MEG, CH), jnp.int32),
          pltpu.VMEM((MEG, CH), jnp.int32),
          pltpu.VMEM((MEG * CH, H), jnp.float32),
          pltpu.VMEM((MEG * CH, H), jnp.float32),
          pltpu.VMEM_SHARED((NA, H), jnp.float32),
          pltpu.SemaphoreType.DMA,
          pltpu.SemaphoreType.DMA,
          pltpu.SemaphoreType.DMA,
          pltpu.SemaphoreType.DMA,
      ],
  )
  return f(mA, dst3, zeros)


def _sc_scatter_h2_body(mB, dst3, part, hp, idx2a, idx2b, rowsa, rowsb,
                        acc, sl0, sl1, sa0, sa1):
  c = lax.axis_index("c")
  s = lax.axis_index("s")
  rows_per_tile = NA // NS
  r0 = s * rows_per_tile
  pltpu.sync_copy(part.at[c, pl.ds(r0, rows_per_tile)],
                  acc.at[pl.ds(r0, rows_per_tile)])
  plsc.subcore_barrier()
  sems = ((sl0, sl1), (sa0, sa1))
  nmeg = (EBH // (MEG * CH)) // NC // NS

  @pl.when(c == 0)
  def _():
    _scatter_accumulate(mB, dst3, acc, s * nmeg, nmeg,
                        (idx2a, idx2b), (rowsa, rowsb), sems, mg_off=MG_A)

  @pl.when(c == 1)
  def _():
    _scatter_accumulate(mB, dst3, acc, 400 + s * nmeg, nmeg,
                        (idx2a, idx2b), (rowsa, rowsb), sems, mg_off=MG_A)

  plsc.subcore_barrier()
  pltpu.sync_copy(acc.at[pl.ds(r0, rows_per_tile)],
                  hp.at[c, pl.ds(r0, rows_per_tile)])


def _sc_scatter_h2(mB, dst3, part):
  f = pl.kernel(
      _sc_scatter_h2_body,
      out_type=jax.ShapeDtypeStruct((NC, NA, H), jnp.float32),
      mesh=_mesh(),
      scratch_types=[
          pltpu.VMEM((MEG, CH), jnp.int32),
          pltpu.VMEM((MEG, CH), jnp.int32),
          pltpu.VMEM((MEG * CH, H), jnp.float32),
          pltpu.VMEM((MEG * CH, H), jnp.float32),
          pltpu.VMEM_SHARED((NA, H), jnp.float32),
          pltpu.SemaphoreType.DMA,
          pltpu.SemaphoreType.DMA,
          pltpu.SemaphoreType.DMA,
          pltpu.SemaphoreType.DMA,
      ],
  )
  return f(mB, dst3, part)


def _gather_pipeline(streams, src1, dst1, outs, idx_s, idx_d, bufs, sem_g,
                     sem_w, e0, n_e):
  w = _worker_id()
  per_w = n_e // NW
  n_ch = per_w // CH
  any_dst = any(use_d for _, _, use_d in streams)

  def fire_gathers(ch, b):
    base = w * per_w + ch * CH
    pltpu.sync_copy(src1.at[pl.ds(e0 + base, CH)], idx_s[b])
    if any_dst:
      pltpu.sync_copy(dst1.at[pl.ds(e0 + base, CH)], idx_d[b])
    for k, (tab, _, use_d) in enumerate(streams):
      idx = idx_d[b] if use_d else idx_s[b]
      pltpu.async_copy(tab.at[idx], bufs[k][b], sem_g[b])

  def wait_gathers(b):
    for k, (tab, _, _) in enumerate(streams):
      _drain(tab.at[pl.ds(0, CH)], bufs[k][b], sem_g[b])

  def fire_wb(ch, b):
    base = w * per_w + ch * CH
    for k in range(len(streams)):
      pltpu.async_copy(bufs[k][b], outs[k].at[pl.ds(base, CH)], sem_w[b])

  def wait_wb(b):
    for k, (tab, _, _) in enumerate(streams):
      _drain(tab.at[pl.ds(0, CH)], bufs[k][b], sem_w[b])

  fire_gathers(0, 0)

  def step(i, _):
    for b in (0, 1):
      ch = 2 * i + b
      wait_gathers(b)
      fire_wb(ch, b)

      @pl.when(ch >= 1)
      def _():
        wait_wb(1 - b)

      @pl.when(ch + 1 < n_ch)
      def _():
        fire_gathers(ch + 1, 1 - b)
    return ()

  lax.fori_loop(0, n_ch // 2, step, (), unroll=False)
  if n_ch % 2 == 1:
    wait_gathers(0)
    fire_wb(n_ch - 1, 0)
    wait_wb(1)
    wait_wb(0)
  else:
    wait_wb(1)


def _sc_gather_edges3(g_tab, ab_tab, c_tab, src1, dst1, e0, n_e):
  def body(g_tab, ab_tab, c_tab, src1, dst1, og, oab, oc,
           i_s0, i_s1, i_d0, i_d1, bg0, bg1, bab0, bab1, bc0, bc1,
           sg0, sg1, sw0, sw1):
    _gather_pipeline(
        [(g_tab, H, False), (ab_tab, H, False), (c_tab, H, True)],
        src1, dst1, (og, oab, oc),
        (i_s0, i_s1), (i_d0, i_d1),
        ((bg0, bg1), (bab0, bab1), (bc0, bc1)),
        (sg0, sg1), (sw0, sw1), e0, n_e)

  f = pl.kernel(
      body,
      out_type=[jax.ShapeDtypeStruct((n_e, H), jnp.int32),
                jax.ShapeDtypeStruct((n_e, H), jnp.int32),
                jax.ShapeDtypeStruct((n_e, H), jnp.float32)],
      mesh=_mesh(),
      scratch_types=[
          pltpu.VMEM((CH,), jnp.int32), pltpu.VMEM((CH,), jnp.int32),
          pltpu.VMEM((CH,), jnp.int32), pltpu.VMEM((CH,), jnp.int32),
          pltpu.VMEM((CH, H), jnp.int32), pltpu.VMEM((CH, H), jnp.int32),
          pltpu.VMEM((CH, H), jnp.int32), pltpu.VMEM((CH, H), jnp.int32),
          pltpu.VMEM((CH, H), jnp.float32), pltpu.VMEM((CH, H), jnp.float32),
          pltpu.SemaphoreType.DMA, pltpu.SemaphoreType.DMA,
          pltpu.SemaphoreType.DMA, pltpu.SemaphoreType.DMA,
      ],
  )
  return f(g_tab, ab_tab, c_tab, src1, dst1)


def _sc_gather_edges1(g_tab, src1, e0, n_e):
  def body(g_tab, src1, og, i_s0, i_s1, bg0, bg1, sg0, sg1, sw0, sw1):
    _gather_pipeline(
        [(g_tab, H, False)], src1, None, (og,),
        (i_s0, i_s1), (None, None), ((bg0, bg1),),
        (sg0, sg1), (sw0, sw1), e0, n_e)

  f = pl.kernel(
      body,
      out_type=jax.ShapeDtypeStruct((n_e, H), jnp.int32),
      mesh=_mesh(),
      scratch_types=[
          pltpu.VMEM((CH,), jnp.int32), pltpu.VMEM((CH,), jnp.int32),
          pltpu.VMEM((CH, H), jnp.int32), pltpu.VMEM((CH, H), jnp.int32),
          pltpu.SemaphoreType.DMA, pltpu.SemaphoreType.DMA,
          pltpu.SemaphoreType.DMA, pltpu.SemaphoreType.DMA,
      ],
  )
  return f(g_tab, src1)


NB = 400
VP = 1024


def _tc_tables_body(widf_ref, tidf_ref, emb_ref, tvec_ref,
                    wz1_ref, wh1_ref, wr_ref, bz_ref, bh_ref, bur_ref,
                    x_ref, tv_ref, ab_ref, c_ref):
  colv = lax.broadcasted_iota(jnp.int32, (NB, VP), 1).astype(jnp.float32)
  ow = jnp.where(colv == widf_ref[...], 1.0, 0.0)
  colt = lax.broadcasted_iota(jnp.int32, (NB, T), 1).astype(jnp.float32)
  ot = jnp.where(colt == tidf_ref[...], 1.0, 0.0)
  x = jnp.dot(ow, emb_ref[...], preferred_element_type=jnp.float32)
  tv = jnp.dot(ot, tvec_ref[...], preferred_element_type=jnp.float32)
  a = jnp.dot(x, wz1_ref[...], preferred_element_type=jnp.float32) + bz_ref[...]
  b = jnp.dot(x, wh1_ref[...], preferred_element_type=jnp.float32) + bh_ref[...]
  c = jnp.dot(x, wr_ref[...], preferred_element_type=jnp.float32) + bur_ref[...]
  x_ref[...] = x
  tv_ref[...] = tv
  ab_ref[...] = _pack2(a, b)
  c_ref[...] = c


def _tc_tables(widf, tidf, emb_p, tvec, Wz1, Wh1, Wr, bz, bh, bur):
  nb_spec = pl.BlockSpec((NB, H), lambda i: (i, 0))
  n1_spec = pl.BlockSpec((NB, 1), lambda i: (i, 0))
  full = lambda shape: pl.BlockSpec(shape, lambda i: tuple(0 for _ in shape))
  return pl.pallas_call(
      _tc_tables_body,
      grid=(N // NB,),
      in_specs=[n1_spec, n1_spec, full((VP, H)), full((T, H)),
                full((H, H)), full((H, H)), full((H, H)),
                full((1, H)), full((1, H)), full((1, H))],
      out_specs=[nb_spec, nb_spec, nb_spec, nb_spec],
      out_shape=[jax.ShapeDtypeStruct((N, H), jnp.float32),
                 jax.ShapeDtypeStruct((N, H), jnp.float32),
                 jax.ShapeDtypeStruct((N, H), jnp.int32),
                 jax.ShapeDtypeStruct((N, H), jnp.float32)],
  )(widf, tidf, emb_p, tvec, Wz1, Wh1, Wr, bz, bh, bur)


GBLK = 632


def _tc_pack_g_body(g_ref, o_ref):
  g = g_ref[...]
  o_ref[...] = _pack2(g[:, :H], g[:, H:])


def _tc_pack_g(g):
  return pl.pallas_call(
      _tc_pack_g_body,
      grid=(NA // GBLK,),
      in_specs=[pl.BlockSpec((GBLK, H2), lambda i: (i, 0))],
      out_specs=pl.BlockSpec((GBLK, H), lambda i: (i, 0)),
      out_shape=jax.ShapeDtypeStruct((NA, H), jnp.int32),
  )(g)


EB = 512


def _pair_swap(x):
  nxt = pltpu.roll(x, x.shape[0] - 1, 0)
  prv = pltpu.roll(x, 1, 0)
  row = lax.broadcasted_iota(jnp.int32, x.shape, 0)
  return jnp.where((row & 1) == 0, nxt, prv)


def _tc_gru_body(m_ref, rm_ref, g_ref, ab_ref, pr_ref,
                 wz2_ref, wh2_ref, ur_ref, mo_ref, rmo_ref):
  gnm, gnrm = _unpack2(g_ref[...])
  a, b = _unpack2(ab_ref[...])
  s = gnm - _pair_swap(m_ref[...])
  arm = gnrm - _pair_swap(rm_ref[...])
  z = jax.nn.sigmoid(
      a + jnp.dot(s, wz2_ref[...],
                  preferred_element_type=jnp.float32))
  mt = jnp.tanh(
      b + jnp.dot(arm, wh2_ref[...],
                  preferred_element_type=jnp.float32))
  mo = (1.0 - z) * s + z * mt
  r = jax.nn.sigmoid(
      pr_ref[...] + jnp.dot(mo, ur_ref[...],
                            preferred_element_type=jnp.float32))
  mo_ref[...] = mo
  rmo_ref[...] = r * mo


def _tc_gru2_body(m_ref, rm_ref, g_ref, ab_ref, wz2_ref, wh2_ref, mo_ref):
  gnm, gnrm = _unpack2(g_ref[...])
  a, b = _unpack2(ab_ref[...])
  s = gnm - _pair_swap(m_ref[...])
  arm = gnrm - _pair_swap(rm_ref[...])
  z = jax.nn.sigmoid(
      a + jnp.dot(s, wz2_ref[...],
                  preferred_element_type=jnp.float32))
  mt = jnp.tanh(
      b + jnp.dot(arm, wh2_ref[...],
                  preferred_element_type=jnp.float32))
  mo_ref[...] = (1.0 - z) * s + z * mt


def _tc_gru2(m, rm, g, ab, Wz2, Wh2, n_e):
  eb_spec = pl.BlockSpec((EB, H), lambda i: (i, 0))
  w_spec = pl.BlockSpec((H, H), lambda i: (0, 0))
  return pl.pallas_call(
      _tc_gru2_body,
      grid=(n_e // EB,),
      in_specs=[eb_spec, eb_spec, eb_spec, eb_spec] + [w_spec] * 2,
      out_specs=eb_spec,
      out_shape=jax.ShapeDtypeStruct((n_e, H), jnp.float32),
  )(m, rm, g, ab, Wz2, Wh2)


def _tc_gru(m, rm, g, ab, pr, Wz2, Wh2, Ur, n_e, mrm_blk0=0):
  eb_spec = pl.BlockSpec((EB, H), lambda i: (i, 0))
  mm_spec = pl.BlockSpec((EB, H), lambda i: (mrm_blk0 + i, 0))
  w_spec = pl.BlockSpec((H, H), lambda i: (0, 0))
  return pl.pallas_call(
      _tc_gru_body,
      grid=(n_e // EB,),
      in_specs=[mm_spec, mm_spec, eb_spec, eb_spec, eb_spec] + [w_spec] * 3,
      out_specs=[eb_spec] * 2,
      out_shape=[jax.ShapeDtypeStruct((n_e, H), jnp.float32)] * 2,
  )(m, rm, g, ab, pr, Wz2, Wh2, Ur)


def _tc_final_body(x_ref, h0_ref, h1_ref, tv_ref, widf_ref, pt_ref,
                   wl1_ref, wl2_ref, blr, wo_ref, bo_ref,
                   uu1_ref, uu2_ref, uu3_ref, bur2, us_ref, bsr,
                   acc_ref):
  pid = pl.program_id(0)

  @pl.when(pid == 0)
  def _():
    for i in range(4):
      acc_ref[i] = 0.0

  h = h0_ref[...] + h1_ref[...]
  x = x_ref[...]
  tv = tv_ref[...]

  qp = jax.nn.relu(
      jnp.dot(h, wl1_ref[...], preferred_element_type=jnp.float32)
      + jnp.dot(tv, wl2_ref[...], preferred_element_type=jnp.float32)
      + blr[...])
  q = jnp.dot(qp, wo_ref[...], preferred_element_type=jnp.float32) + bo_ref[...]

  pp = jax.nn.relu(
      jnp.dot(x, uu1_ref[...], preferred_element_type=jnp.float32)
      + jnp.dot(h, uu2_ref[...], preferred_element_type=jnp.float32)
      + jnp.dot(tv, uu3_ref[...], preferred_element_type=jnp.float32)
      + bur2[...])
  p = jnp.dot(pp, us_ref[...], preferred_element_type=jnp.float32) + bsr[...]

  pt = pt_ref[...]
  p_loss = jnp.sum(jnp.maximum(p, 0.0) - p * pt
                   + jnp.log(1.0 + jnp.exp(-jnp.abs(p))))
  p_hit = jnp.sum(jnp.where((p > 0.0) == (pt > 0.5), 1.0, 0.0))

  widf = widf_ref[...]
  coli = lax.broadcasted_iota(jnp.int32, q.shape, 1)
  col = coli.astype(jnp.float32)
  onehot = jnp.where(col == widf, 1.0, 0.0)
  q_sel = jnp.sum(q * onehot, axis=1, keepdims=True)
  q_max = jnp.max(q, axis=1, keepdims=True)
  lse = jnp.log(jnp.sum(jnp.exp(q - q_max), axis=1, keepdims=True)) + q_max
  q_loss = jnp.sum(lse - q_sel)

  am = jnp.min(jnp.where(q == q_max, coli, VP), axis=1, keepdims=True)
  q_hit = jnp.sum(jnp.where(am.astype(jnp.float32) == widf, 1.0, 0.0))

  acc_ref[0] += q_loss
  acc_ref[1] += p_loss
  acc_ref[2] += q_hit
  acc_ref[3] += p_hit


def _tc_final(x, h2, tv, widf, pt, Wl, bl, Wo_p, bo_p, Uu, bu, Us, bs):
  nb_spec = pl.BlockSpec((NB, H), lambda i: (i, 0))
  n1_spec = pl.BlockSpec((NB, 1), lambda i: (i, 0))
  full = lambda shape: pl.BlockSpec(shape, lambda i: tuple(0 for _ in shape))
  return pl.pallas_call(
      _tc_final_body,
      grid=(N // NB,),
      in_specs=[nb_spec, nb_spec, nb_spec, nb_spec, n1_spec, n1_spec,
                full((H, H)), full((H, H)), full((1, H)),
                full((H, VP)), full((1, VP)),
                full((H, H)), full((H, H)), full((H, H)), full((1, H)),
                full((H, 1)), full((1, 1))],
      out_specs=pl.BlockSpec(memory_space=pltpu.MemorySpace.SMEM),
      out_shape=jax.ShapeDtypeStruct((4,), jnp.float32),
  )(x, h2[0], h2[1], tv, widf, pt,
    Wl[:H], Wl[H:], bl.reshape(1, H), Wo_p, bo_p,
    Uu[:H], Uu[H:2 * H], Uu[2 * H:], bu.reshape(1, H), Us, bs.reshape(1, 1))


def kernel(wid, edge_index, tree_id, tree_vec, m0, rm0, p_targets, emb,
           Wz, bz, Wh, bh, Wr, Ur, bur, Wl, bl, Wo, bo, Uu, bu, Us, bs):
  src = edge_index[0].astype(jnp.int32)
  dst = edge_index[1].astype(jnp.int32)
  dst3 = dst.reshape(E // (MEG * CH), MEG, CH)
  zeros_n = jnp.zeros((NA, H), jnp.float32)
  widf = wid.astype(jnp.float32).reshape(N, 1)
  tidf = tree_id.astype(jnp.float32).reshape(N, 1)
  emb_p = jnp.concatenate([emb, jnp.zeros((VP - V, H), jnp.float32)])

  x_n, tv_n, ab_n, c_n = _tc_tables(
      widf, tidf, emb_p, tree_vec, Wz[:H], Wh[:H], Wr,
      bz.reshape(1, H), bh.reshape(1, H), bur.reshape(1, H))

  p1 = _sc_scatter2_part1(m0, rm0, dst3, zeros_n)
  g1 = _sc_scatter2_part2(m0, rm0, dst3, p1, tab_mg0=MG_A)
  g1p = _tc_pack_g(g1)
  ggA, abA, prA = _sc_gather_edges3(g1p, ab_n, c_n, src, dst, 0, EA)
  ggB, abB, prB = _sc_gather_edges3(g1p, ab_n, c_n, src, dst, EA, EBH)
  m1A, rm1A = _tc_gru(m0, rm0, ggA, abA, prA, Wz[H:], Wh[H:], Ur, EA)
  m1B, rm1B = _tc_gru(m0, rm0, ggB, abB, prB, Wz[H:], Wh[H:], Ur, EBH,
                      mrm_blk0=EA // EB)

  p2 = _sc_scatter2_part1(m1A, rm1A, dst3, zeros_n)
  g2 = _sc_scatter2_part2(m1B, rm1B, dst3, p2)
  g2p = _tc_pack_g(g2)
  gg2A = _sc_gather_edges1(g2p, src, 0, EA)
  gg2B = _sc_gather_edges1(g2p, src, EA, EBH)
  m2A = _tc_gru2(m1A, rm1A, gg2A, abA, Wz[H:], Wh[H:], EA)
  m2B = _tc_gru2(m1B, rm1B, gg2B, abB, Wz[H:], Wh[H:], EBH)

  hp = _sc_scatter_h1(m2A, dst3, zeros_n)
  h2 = _sc_scatter_h2(m2B, dst3, hp)

  Wo_p = jnp.concatenate([Wo, jnp.zeros((H, VP - V), jnp.float32)], axis=1)
  bo_p = jnp.concatenate([bo, jnp.full((VP - V,), -1e9, jnp.float32)])
  ptf = p_targets.astype(jnp.float32).reshape(N, 1)

  acc = _tc_final(x_n, h2[:, :N], tv_n, widf, ptf,
                  Wl, bl, Wo_p, bo_p.reshape(1, VP), Uu, bu, Us, bs)

  n_trees = float(T)
  q_loss = acc[0] / n_trees
  p_loss = acc[1] / n_trees
  q_acc = acc[2] / float(N)
  p_acc = acc[3] / float(N)
  return q_loss, p_loss, q_acc, p_acc


# speedup vs baseline: 3.6395x; 1.0555x over previous
import jax
import jax.numpy as jnp
from jax import lax
from jax.experimental import pallas as pl
from jax.experimental.pallas import tpu as pltpu
from jax.experimental.pallas import tpu_sc as plsc

N = 10000
E = 320000
H = 128
H2 = 2 * H
HQ = H // 2
L = 128
V = 1000
T = 256

NC = 2
NS = 16
NW = NC * NS
CH = 80
MEG = 2

NA = 10112

SL_N = (128000, 102400, 89600)
SL_O = (0, 128000, 230400)
SL_MG = (800, 640, 560)
SL_MGO = (0, 800, 1440)

_mesh = lambda: plsc.VectorSubcoreMesh(
    core_axis_name="c", subcore_axis_name="s", num_cores=NC, num_subcores=NS)


def _worker_id():
  return lax.axis_index("s") * NC + lax.axis_index("c")


def _pack2(a, b):
  au = lax.bitcast_convert_type(a.astype(jnp.bfloat16), jnp.uint16)
  bu = lax.bitcast_convert_type(b.astype(jnp.bfloat16), jnp.uint16)
  word = (bu.astype(jnp.uint32) << 16) | au.astype(jnp.uint32)
  return lax.bitcast_convert_type(word, jnp.int32)


def _unpack2(w):
  wu = lax.bitcast_convert_type(w, jnp.uint32)
  a = lax.bitcast_convert_type(wu << 16, jnp.float32)
  b = lax.bitcast_convert_type(wu & jnp.uint32(0xFFFF0000), jnp.float32)
  return a, b


def _drain(dummy_src, dst, sem):
  pltpu.make_async_copy(dummy_src, dst, sem).wait()


def _scatter_accumulate(tab, dst3, acc, tile_mg0, n_meg, idx2, rows, sems,
                        mg_off=0, tab_mg_off=None):
  if tab_mg_off is None:
    tab_mg_off = 0
  sem_l, sem_a = sems

  def load(mi, b):
    mg = tile_mg0 + mi
    pltpu.async_copy(dst3.at[mg + mg_off], idx2[b], sem_l[b])
    pltpu.async_copy(tab.at[pl.ds((mg + tab_mg_off) * MEG * CH, MEG * CH)],
                     rows[b], sem_l[b])

  def wait_load(b):
    _drain(dst3.at[0], idx2[b], sem_l[b])
    _drain(tab.at[pl.ds(0, MEG * CH)], rows[b], sem_l[b])

  def fire_adds(b):
    for j in range(MEG):
      pltpu.async_copy(rows[b].at[pl.ds(j * CH, CH)], acc.at[idx2[b].at[j]],
                       sem_a[b], add=True)

  def wait_adds(b):
    _drain(tab.at[pl.ds(0, MEG * CH)], rows[b], sem_a[b])

  load(0, 0)

  def step(mi, _):
    for b in (0, 1):
      m = 2 * mi + b

      @pl.when(m < n_meg)
      def _():
        wait_load(b)
        fire_adds(b)

        @pl.when(m + 1 < n_meg)
        def _():
          load(m + 1, 1 - b)

        wait_adds(b)
    return ()

  lax.fori_loop(0, (n_meg + 1) // 2, step, (), unroll=False)


def _acc_zero_and_barrier(zeros, acc, r0, n_rows):
  pltpu.sync_copy(zeros.at[pl.ds(r0, n_rows)], acc.at[pl.ds(r0, n_rows)])
  plsc.subcore_barrier()


_SC_SCRATCH = lambda: [
    pltpu.VMEM((MEG, CH), jnp.int32),
    pltpu.VMEM((MEG, CH), jnp.int32),
    pltpu.VMEM((MEG * CH, H), jnp.float32),
    pltpu.VMEM((MEG * CH, H), jnp.float32),
    pltpu.VMEM_SHARED((NA, H), jnp.float32),
    pltpu.SemaphoreType.DMA,
    pltpu.SemaphoreType.DMA,
    pltpu.SemaphoreType.DMA,
    pltpu.SemaphoreType.DMA,
]


def _dump_packed(c, acc, g, r0, n_rows):
  @pl.when(c == 0)
  def _():
    pltpu.sync_copy(acc.at[pl.ds(r0, n_rows)],
                    g.at[pl.ds(r0, n_rows), pl.ds(0, H)])

  @pl.when(c == 1)
  def _():
    pltpu.sync_copy(acc.at[pl.ds(r0, n_rows)],
                    g.at[pl.ds(r0, n_rows), pl.ds(H, H)])


def _sc_scatter2_full(m, rm, dst3, zeros):
  def body(m, rm, dst3, zeros, g, idx2a, idx2b, rowsa, rowsb, acc,
           sl0, sl1, sa0, sa1):
    c = lax.axis_index("c")
    s = lax.axis_index("s")
    rows_per_tile = NA // NS
    r0 = s * rows_per_tile
    _acc_zero_and_barrier(zeros, acc, r0, rows_per_tile)
    per_tile = (E // (MEG * CH)) // NS
    sems = ((sl0, sl1), (sa0, sa1))

    @pl.when(c == 0)
    def _():
      _scatter_accumulate(m, dst3, acc, s * per_tile, per_tile,
                          (idx2a, idx2b), (rowsa, rowsb), sems)

    @pl.when(c == 1)
    def _():
      _scatter_accumulate(rm, dst3, acc, s * per_tile, per_tile,
                          (idx2a, idx2b), (rowsa, rowsb), sems)

    plsc.subcore_barrier()
    _dump_packed(c, acc, g, r0, rows_per_tile)

  f = pl.kernel(
      body,
      out_type=jax.ShapeDtypeStruct((NA, H2), jnp.float32),
      mesh=_mesh(),
      scratch_types=_SC_SCRATCH(),
  )
  return f(m, rm, dst3, zeros)


def _sc_scatter2_phase(m_sl, rm_sl, dst3, init, sl_i):
  first = sl_i == 0
  last = sl_i == len(SL_N) - 1
  n_tile = SL_MG[sl_i] // NS
  mg0 = SL_MGO[sl_i]

  def body(m, rm, dst3, init, out, idx2a, idx2b, rowsa, rowsb, acc,
           sl0, sl1, sa0, sa1):
    c = lax.axis_index("c")
    s = lax.axis_index("s")
    rows_per_tile = NA // NS
    r0 = s * rows_per_tile
    if first:
      _acc_zero_and_barrier(init, acc, r0, rows_per_tile)
    else:
      pltpu.sync_copy(init.at[c, pl.ds(r0, rows_per_tile)],
                      acc.at[pl.ds(r0, rows_per_tile)])
      plsc.subcore_barrier()
    sems = ((sl0, sl1), (sa0, sa1))

    @pl.when(c == 0)
    def _():
      _scatter_accumulate(m, dst3, acc, s * n_tile, n_tile,
                          (idx2a, idx2b), (rowsa, rowsb), sems, mg_off=mg0)

    @pl.when(c == 1)
    def _():
      _scatter_accumulate(rm, dst3, acc, s * n_tile, n_tile,
                          (idx2a, idx2b), (rowsa, rowsb), sems, mg_off=mg0)

    plsc.subcore_barrier()
    if last:
      _dump_packed(c, acc, out, r0, rows_per_tile)
    else:
      pltpu.sync_copy(acc.at[pl.ds(r0, rows_per_tile)],
                      out.at[c, pl.ds(r0, rows_per_tile)])

  out_t = (jax.ShapeDtypeStruct((NA, H2), jnp.float32) if last
           else jax.ShapeDtypeStruct((NC, NA, H), jnp.float32))
  f = pl.kernel(body, out_type=out_t, mesh=_mesh(),
                scratch_types=_SC_SCRATCH())
  return f(m_sl, rm_sl, dst3, init)


_H_SPLIT = ((25, 25), (20, 20), (18, 17))


def _sc_scatter_h_phase(m_sl, dst3, init, sl_i):
  first = sl_i == 0
  n0, n1 = _H_SPLIT[sl_i]
  mg0 = SL_MGO[sl_i]

  def body(m, dst3, init, out, idx2a, idx2b, rowsa, rowsb, acc,
           sl0, sl1, sa0, sa1):
    c = lax.axis_index("c")
    s = lax.axis_index("s")
    rows_per_tile = NA // NS
    r0 = s * rows_per_tile
    if first:
      _acc_zero_and_barrier(init, acc, r0, rows_per_tile)
    else:
      pltpu.sync_copy(init.at[c, pl.ds(r0, rows_per_tile)],
                      acc.at[pl.ds(r0, rows_per_tile)])
      plsc.subcore_barrier()
    sems = ((sl0, sl1), (sa0, sa1))

    @pl.when(c == 0)
    def _():
      _scatter_accumulate(m, dst3, acc, s * n0, n0,
                          (idx2a, idx2b), (rowsa, rowsb), sems, mg_off=mg0)

    @pl.when(c == 1)
    def _():
      _scatter_accumulate(m, dst3, acc, NS * n0 + s * n1, n1,
                          (idx2a, idx2b), (rowsa, rowsb), sems, mg_off=mg0)

    plsc.subcore_barrier()
    pltpu.sync_copy(acc.at[pl.ds(r0, rows_per_tile)],
                    out.at[c, pl.ds(r0, rows_per_tile)])

  f = pl.kernel(body, out_type=jax.ShapeDtypeStruct((NC, NA, H), jnp.float32),
                mesh=_mesh(), scratch_types=_SC_SCRATCH())
  return f(m_sl, dst3, init)


def _gather_pipeline(streams, src1, dst1, outs, idx_s, idx_d, bufs, sem_g,
                     sem_w, e0, n_e):
  w = _worker_id()
  per_w = n_e // NW
  n_ch = per_w // CH
  any_dst = any(use_d for _, _, use_d in streams)

  def fire_gathers(ch, b):
    base = w * per_w + ch * CH
    pltpu.sync_copy(src1.at[pl.ds(e0 + base, CH)], idx_s[b])
    if any_dst:
      pltpu.sync_copy(dst1.at[pl.ds(e0 + base, CH)], idx_d[b])
    for k, (tab, _, use_d) in enumerate(streams):
      idx = idx_d[b] if use_d else idx_s[b]
      pltpu.async_copy(tab.at[idx], bufs[k][b], sem_g[b])

  def wait_gathers(b):
    for k, (tab, _, _) in enumerate(streams):
      _drain(tab.at[pl.ds(0, CH)], bufs[k][b], sem_g[b])

  def fire_wb(ch, b):
    base = w * per_w + ch * CH
    for k in range(len(streams)):
      pltpu.async_copy(bufs[k][b], outs[k].at[pl.ds(base, CH)], sem_w[b])

  def wait_wb(b):
    for k, (tab, _, _) in enumerate(streams):
      _drain(tab.at[pl.ds(0, CH)], bufs[k][b], sem_w[b])

  fire_gathers(0, 0)

  def step(i, _):
    for b in (0, 1):
      ch = 2 * i + b
      wait_gathers(b)
      fire_wb(ch, b)

      @pl.when(ch >= 1)
      def _():
        wait_wb(1 - b)

      @pl.when(ch + 1 < n_ch)
      def _():
        fire_gathers(ch + 1, 1 - b)
    return ()

  lax.fori_loop(0, n_ch // 2, step, (), unroll=False)
  if n_ch % 2 == 1:
    wait_gathers(0)
    fire_wb(n_ch - 1, 0)
    wait_wb(1)
    wait_wb(0)
  else:
    wait_wb(1)


def _sc_gather_edges3(g_tab, ab_tab, c_tab, src1, dst1, e0, n_e):
  def body(g_tab, ab_tab, c_tab, src1, dst1, og, oab, oc,
           i_s0, i_s1, i_d0, i_d1, bg0, bg1, bab0, bab1, bc0, bc1,
           sg0, sg1, sw0, sw1):
    _gather_pipeline(
        [(g_tab, H, False), (ab_tab, H, False), (c_tab, H, True)],
        src1, dst1, (og, oab, oc),
        (i_s0, i_s1), (i_d0, i_d1),
        ((bg0, bg1), (bab0, bab1), (bc0, bc1)),
        (sg0, sg1), (sw0, sw1), e0, n_e)

  f = pl.kernel(
      body,
      out_type=[jax.ShapeDtypeStruct((n_e, H), jnp.int32),
                jax.ShapeDtypeStruct((n_e, H), jnp.int32),
                jax.ShapeDtypeStruct((n_e, H), jnp.float32)],
      mesh=_mesh(),
      scratch_types=[
          pltpu.VMEM((CH,), jnp.int32), pltpu.VMEM((CH,), jnp.int32),
          pltpu.VMEM((CH,), jnp.int32), pltpu.VMEM((CH,), jnp.int32),
          pltpu.VMEM((CH, H), jnp.int32), pltpu.VMEM((CH, H), jnp.int32),
          pltpu.VMEM((CH, H), jnp.int32), pltpu.VMEM((CH, H), jnp.int32),
          pltpu.VMEM((CH, H), jnp.float32), pltpu.VMEM((CH, H), jnp.float32),
          pltpu.SemaphoreType.DMA, pltpu.SemaphoreType.DMA,
          pltpu.SemaphoreType.DMA, pltpu.SemaphoreType.DMA,
      ],
  )
  return f(g_tab, ab_tab, c_tab, src1, dst1)


def _sc_gather_edges1(g_tab, src1, e0, n_e):
  def body(g_tab, src1, og, i_s0, i_s1, bg0, bg1, sg0, sg1, sw0, sw1):
    _gather_pipeline(
        [(g_tab, H, False)], src1, None, (og,),
        (i_s0, i_s1), (None, None), ((bg0, bg1),),
        (sg0, sg1), (sw0, sw1), e0, n_e)

  f = pl.kernel(
      body,
      out_type=jax.ShapeDtypeStruct((n_e, H), jnp.int32),
      mesh=_mesh(),
      scratch_types=[
          pltpu.VMEM((CH,), jnp.int32), pltpu.VMEM((CH,), jnp.int32),
          pltpu.VMEM((CH, H), jnp.int32), pltpu.VMEM((CH, H), jnp.int32),
          pltpu.SemaphoreType.DMA, pltpu.SemaphoreType.DMA,
          pltpu.SemaphoreType.DMA, pltpu.SemaphoreType.DMA,
      ],
  )
  return f(g_tab, src1)


NB = 400
VP = 1024


def _tc_tables_body(widf_ref, tidf_ref, emb_ref, tvec_ref,
                    wz1_ref, wh1_ref, wr_ref, bz_ref, bh_ref, bur_ref,
                    x_ref, tv_ref, ab_ref, c_ref):
  colv = lax.broadcasted_iota(jnp.int32, (NB, VP), 1).astype(jnp.float32)
  ow = jnp.where(colv == widf_ref[...], 1.0, 0.0)
  colt = lax.broadcasted_iota(jnp.int32, (NB, T), 1).astype(jnp.float32)
  ot = jnp.where(colt == tidf_ref[...], 1.0, 0.0)
  x = jnp.dot(ow, emb_ref[...], preferred_element_type=jnp.float32)
  tv = jnp.dot(ot, tvec_ref[...], preferred_element_type=jnp.float32)
  a = jnp.dot(x, wz1_ref[...], preferred_element_type=jnp.float32) + bz_ref[...]
  b = jnp.dot(x, wh1_ref[...], preferred_element_type=jnp.float32) + bh_ref[...]
  c = jnp.dot(x, wr_ref[...], preferred_element_type=jnp.float32) + bur_ref[...]
  x_ref[...] = x
  tv_ref[...] = tv
  ab_ref[...] = _pack2(a, b)
  c_ref[...] = c


def _tc_tables(widf, tidf, emb_p, tvec, Wz1, Wh1, Wr, bz, bh, bur):
  nb_spec = pl.BlockSpec((NB, H), lambda i: (i, 0))
  n1_spec = pl.BlockSpec((NB, 1), lambda i: (i, 0))
  full = lambda shape: pl.BlockSpec(shape, lambda i: tuple(0 for _ in shape))
  return pl.pallas_call(
      _tc_tables_body,
      grid=(N // NB,),
      in_specs=[n1_spec, n1_spec, full((VP, H)), full((T, H)),
                full((H, H)), full((H, H)), full((H, H)),
                full((1, H)), full((1, H)), full((1, H))],
      out_specs=[nb_spec, nb_spec, nb_spec, nb_spec],
      out_shape=[jax.ShapeDtypeStruct((N, H), jnp.float32),
                 jax.ShapeDtypeStruct((N, H), jnp.float32),
                 jax.ShapeDtypeStruct((N, H), jnp.int32),
                 jax.ShapeDtypeStruct((N, H), jnp.float32)],
  )(widf, tidf, emb_p, tvec, Wz1, Wh1, Wr, bz, bh, bur)


GBLK = 632


def _tc_pack_g_body(g_ref, o_ref):
  g = g_ref[...]
  o_ref[...] = _pack2(g[:, :H], g[:, H:])


def _tc_pack_g(g):
  return pl.pallas_call(
      _tc_pack_g_body,
      grid=(NA // GBLK,),
      in_specs=[pl.BlockSpec((GBLK, H2), lambda i: (i, 0))],
      out_specs=pl.BlockSpec((GBLK, H), lambda i: (i, 0)),
      out_shape=jax.ShapeDtypeStruct((NA, H), jnp.int32),
  )(g)


EB = 512


def _pair_swap(x):
  nxt = pltpu.roll(x, x.shape[0] - 1, 0)
  prv = pltpu.roll(x, 1, 0)
  row = lax.broadcasted_iota(jnp.int32, x.shape, 0)
  return jnp.where((row & 1) == 0, nxt, prv)


def _tc_gru_body(m_ref, rm_ref, g_ref, ab_ref, pr_ref,
                 wz2_ref, wh2_ref, ur_ref, mo_ref, rmo_ref):
  gnm, gnrm = _unpack2(g_ref[...])
  a, b = _unpack2(ab_ref[...])
  s = gnm - _pair_swap(m_ref[...])
  arm = gnrm - _pair_swap(rm_ref[...])
  z = jax.nn.sigmoid(
      a + jnp.dot(s, wz2_ref[...],
                  preferred_element_type=jnp.float32))
  mt = jnp.tanh(
      b + jnp.dot(arm, wh2_ref[...],
                  preferred_element_type=jnp.float32))
  mo = (1.0 - z) * s + z * mt
  r = jax.nn.sigmoid(
      pr_ref[...] + jnp.dot(mo, ur_ref[...],
                            preferred_element_type=jnp.float32))
  mo_ref[...] = mo
  rmo_ref[...] = r * mo


def _tc_gru2_body(m_ref, rm_ref, g_ref, ab_ref, wz2_ref, wh2_ref, mo_ref):
  gnm, gnrm = _unpack2(g_ref[...])
  a, b = _unpack2(ab_ref[...])
  s = gnm - _pair_swap(m_ref[...])
  arm = gnrm - _pair_swap(rm_ref[...])
  z = jax.nn.sigmoid(
      a + jnp.dot(s, wz2_ref[...],
                  preferred_element_type=jnp.float32))
  mt = jnp.tanh(
      b + jnp.dot(arm, wh2_ref[...],
                  preferred_element_type=jnp.float32))
  mo_ref[...] = (1.0 - z) * s + z * mt


def _tc_gru2(m, rm, g, ab, Wz2, Wh2, n_e):
  eb_spec = pl.BlockSpec((EB, H), lambda i: (i, 0))
  w_spec = pl.BlockSpec((H, H), lambda i: (0, 0))
  return pl.pallas_call(
      _tc_gru2_body,
      grid=(n_e // EB,),
      in_specs=[eb_spec, eb_spec, eb_spec, eb_spec] + [w_spec] * 2,
      out_specs=eb_spec,
      out_shape=jax.ShapeDtypeStruct((n_e, H), jnp.float32),
  )(m, rm, g, ab, Wz2, Wh2)


def _tc_gru(m, rm, g, ab, pr, Wz2, Wh2, Ur, n_e, mrm_blk0=0):
  eb_spec = pl.BlockSpec((EB, H), lambda i: (i, 0))
  mm_spec = pl.BlockSpec((EB, H), lambda i: (mrm_blk0 + i, 0))
  w_spec = pl.BlockSpec((H, H), lambda i: (0, 0))
  return pl.pallas_call(
      _tc_gru_body,
      grid=(n_e // EB,),
      in_specs=[mm_spec, mm_spec, eb_spec, eb_spec, eb_spec] + [w_spec] * 3,
      out_specs=[eb_spec] * 2,
      out_shape=[jax.ShapeDtypeStruct((n_e, H), jnp.float32)] * 2,
  )(m, rm, g, ab, pr, Wz2, Wh2, Ur)


def _tc_final_body(x_ref, h0_ref, h1_ref, tv_ref, widf_ref, pt_ref,
                   wl1_ref, wl2_ref, blr, wo_ref, bo_ref,
                   uu1_ref, uu2_ref, uu3_ref, bur2, us_ref, bsr,
                   acc_ref):
  pid = pl.program_id(0)

  @pl.when(pid == 0)
  def _():
    for i in range(4):
      acc_ref[i] = 0.0

  h = h0_ref[...] + h1_ref[...]
  x = x_ref[...]
  tv = tv_ref[...]

  qp = jax.nn.relu(
      jnp.dot(h, wl1_ref[...], preferred_element_type=jnp.float32)
      + jnp.dot(tv, wl2_ref[...], preferred_element_type=jnp.float32)
      + blr[...])
  q = jnp.dot(qp, wo_ref[...], preferred_element_type=jnp.float32) + bo_ref[...]

  pp = jax.nn.relu(
      jnp.dot(x, uu1_ref[...], preferred_element_type=jnp.float32)
      + jnp.dot(h, uu2_ref[...], preferred_element_type=jnp.float32)
      + jnp.dot(tv, uu3_ref[...], preferred_element_type=jnp.float32)
      + bur2[...])
  p = jnp.dot(pp, us_ref[...], preferred_element_type=jnp.float32) + bsr[...]

  pt = pt_ref[...]
  p_loss = jnp.sum(jnp.maximum(p, 0.0) - p * pt
                   + jnp.log(1.0 + jnp.exp(-jnp.abs(p))))
  p_hit = jnp.sum(jnp.where((p > 0.0) == (pt > 0.5), 1.0, 0.0))

  widf = widf_ref[...]
  coli = lax.broadcasted_iota(jnp.int32, q.shape, 1)
  col = coli.astype(jnp.float32)
  onehot = jnp.where(col == widf, 1.0, 0.0)
  q_sel = jnp.sum(q * onehot, axis=1, keepdims=True)
  q_max = jnp.max(q, axis=1, keepdims=True)
  lse = jnp.log(jnp.sum(jnp.exp(q - q_max), axis=1, keepdims=True)) + q_max
  q_loss = jnp.sum(lse - q_sel)

  am = jnp.min(jnp.where(q == q_max, coli, VP), axis=1, keepdims=True)
  q_hit = jnp.sum(jnp.where(am.astype(jnp.float32) == widf, 1.0, 0.0))

  acc_ref[0] += q_loss
  acc_ref[1] += p_loss
  acc_ref[2] += q_hit
  acc_ref[3] += p_hit


def _tc_final(x, h2, tv, widf, pt, Wl, bl, Wo_p, bo_p, Uu, bu, Us, bs):
  nb_spec = pl.BlockSpec((NB, H), lambda i: (i, 0))
  n1_spec = pl.BlockSpec((NB, 1), lambda i: (i, 0))
  full = lambda shape: pl.BlockSpec(shape, lambda i: tuple(0 for _ in shape))
  return pl.pallas_call(
      _tc_final_body,
      grid=(N // NB,),
      in_specs=[nb_spec, nb_spec, nb_spec, nb_spec, n1_spec, n1_spec,
                full((H, H)), full((H, H)), full((1, H)),
                full((H, VP)), full((1, VP)),
                full((H, H)), full((H, H)), full((H, H)), full((1, H)),
                full((H, 1)), full((1, 1))],
      out_specs=pl.BlockSpec(memory_space=pltpu.MemorySpace.SMEM),
      out_shape=jax.ShapeDtypeStruct((4,), jnp.float32),
  )(x, h2[0], h2[1], tv, widf, pt,
    Wl[:H], Wl[H:], bl.reshape(1, H), Wo_p, bo_p,
    Uu[:H], Uu[H:2 * H], Uu[2 * H:], bu.reshape(1, H), Us, bs.reshape(1, 1))


def kernel(wid, edge_index, tree_id, tree_vec, m0, rm0, p_targets, emb,
           Wz, bz, Wh, bh, Wr, Ur, bur, Wl, bl, Wo, bo, Uu, bu, Us, bs):
  src = edge_index[0].astype(jnp.int32)
  dst = edge_index[1].astype(jnp.int32)
  dst3 = dst.reshape(E // (MEG * CH), MEG, CH)
  zeros_n = jnp.zeros((NA, H), jnp.float32)
  widf = wid.astype(jnp.float32).reshape(N, 1)
  tidf = tree_id.astype(jnp.float32).reshape(N, 1)
  emb_p = jnp.concatenate([emb, jnp.zeros((VP - V, H), jnp.float32)])

  x_n, tv_n, ab_n, c_n = _tc_tables(
      widf, tidf, emb_p, tree_vec, Wz[:H], Wh[:H], Wr,
      bz.reshape(1, H), bh.reshape(1, H), bur.reshape(1, H))

  g1 = _sc_scatter2_full(m0, rm0, dst3, zeros_n)
  g1p = _tc_pack_g(g1)
  ggs, abs_, prs, m1s, rm1s = [], [], [], [], []
  for i in range(3):
    gg, ab_e, pr_e = _sc_gather_edges3(g1p, ab_n, c_n, src, dst,
                                       SL_O[i], SL_N[i])
    ggs.append(gg); abs_.append(ab_e); prs.append(pr_e)
  for i in range(3):
    m1, rm1 = _tc_gru(m0, rm0, ggs[i], abs_[i], prs[i], Wz[H:], Wh[H:], Ur,
                      SL_N[i], mrm_blk0=SL_O[i] // EB)
    m1s.append(m1); rm1s.append(rm1)

  acc = zeros_n
  for i in range(3):
    acc = _sc_scatter2_phase(m1s[i], rm1s[i], dst3, acc, i)
  g2p = _tc_pack_g(acc)
  m2s = []
  gg2s = [_sc_gather_edges1(g2p, src, SL_O[i], SL_N[i]) for i in range(3)]
  for i in range(3):
    m2s.append(_tc_gru2(m1s[i], rm1s[i], gg2s[i], abs_[i],
                        Wz[H:], Wh[H:], SL_N[i]))

  hacc = zeros_n
  for i in range(3):
    hacc = _sc_scatter_h_phase(m2s[i], dst3, hacc, i)
  h2 = hacc

  Wo_p = jnp.concatenate([Wo, jnp.zeros((H, VP - V), jnp.float32)], axis=1)
  bo_p = jnp.concatenate([bo, jnp.full((VP - V,), -1e9, jnp.float32)])
  ptf = p_targets.astype(jnp.float32).reshape(N, 1)

  acc = _tc_final(x_n, h2[:, :N], tv_n, widf, ptf,
                  Wl, bl, Wo_p, bo_p.reshape(1, VP), Uu, bu, Us, bs)

  n_trees = float(T)
  q_loss = acc[0] / n_trees
  p_loss = acc[1] / n_trees
  q_acc = acc[2] / float(N)
  p_acc = acc[3] / float(N)
  return q_loss, p_loss, q_acc, p_acc
